# baseline (speedup 1.0000x reference)
import functools

import jax
import jax.numpy as jnp
import numpy as np
from jax import lax
from jax.experimental import pallas as pl
from jax.experimental.pallas import tpu as pltpu
from jax.experimental.pallas import tpu_sc as plsc

N = 100000
E = 1600000
HID = 32
G = 5
HALF = N // 2
AROWS = 50016
NC, NS, LANES = 2, 16, 16
NW = NC * NS

BLK = 400
SUB = 80
NSUB = BLK // SUB
EPT_G = E // NW
NBLK_G = EPT_G // BLK
SUBB = 128
NBATCH = E // SUBB
QMAX = -(-NBATCH // NS)
ZCH = 521
RPT = AROWS // NS
NZ = RPT // ZCH

BE = 2000
NB = 400

_SC_PARAMS = pltpu.CompilerParams(use_tc_tiling_on_sc=False)


def _silu(v):
    return v * jax.nn.sigmoid(v)


def _gather_body(h_hbm, xc_hbm, row_hbm, col_hbm,
                 hrow_hbm, hcol_hbm, xr_hbm, xcv_hbm,
                 idxr, idxc, hrow_v, hcol_v, xr_v, xcv_v,
                 sem0, sem1, sem2, sem3):
    c = lax.axis_index("c")
    s = lax.axis_index("s")
    wid = s * NC + c

    def blk(j, carry):
        base = wid * EPT_G + j * BLK
        pltpu.sync_copy(row_hbm.at[pl.ds(base, BLK)], idxr)
        pltpu.sync_copy(col_hbm.at[pl.ds(base, BLK)], idxc)
        cps = []
        for t in range(NSUB):
            sl = pl.ds(t * SUB, SUB)
            cps.append(pltpu.async_copy(h_hbm.at[idxr.at[sl]], hrow_v.at[sl], sem0))
            cps.append(pltpu.async_copy(h_hbm.at[idxc.at[sl]], hcol_v.at[sl], sem1))
            cps.append(pltpu.async_copy(xc_hbm.at[idxr.at[sl]], xr_v.at[sl], sem2))
            cps.append(pltpu.async_copy(xc_hbm.at[idxc.at[sl]], xcv_v.at[sl], sem3))
        for cp in cps:
            cp.wait()
        pltpu.sync_copy(hrow_v, hrow_hbm.at[pl.ds(base, BLK)])
        pltpu.sync_copy(hcol_v, hcol_hbm.at[pl.ds(base, BLK)])
        pltpu.sync_copy(xr_v, xr_hbm.at[pl.ds(base, BLK)])
        pltpu.sync_copy(xcv_v, xcv_hbm.at[pl.ds(base, BLK)])
        return carry

    lax.fori_loop(0, NBLK_G, blk, 0)


def _gather_call(h, xc8, row1d, col1d):
    mesh = plsc.VectorSubcoreMesh(core_axis_name="c", subcore_axis_name="s",
                                  num_cores=NC, num_subcores=NS)
    f = pl.kernel(
        _gather_body,
        out_type=[jax.ShapeDtypeStruct((E, HID), jnp.float32),
                  jax.ShapeDtypeStruct((E, HID), jnp.float32),
                  jax.ShapeDtypeStruct((E, 8), jnp.float32),
                  jax.ShapeDtypeStruct((E, 8), jnp.float32)],
        mesh=mesh,
        scratch_types=[
            pltpu.VMEM((BLK,), jnp.int32),
            pltpu.VMEM((BLK,), jnp.int32),
            pltpu.VMEM((BLK, HID), jnp.float32),
            pltpu.VMEM((BLK, HID), jnp.float32),
            pltpu.VMEM((BLK, 8), jnp.float32),
            pltpu.VMEM((BLK, 8), jnp.float32),
            pltpu.SemaphoreType.DMA,
            pltpu.SemaphoreType.DMA,
            pltpu.SemaphoreType.DMA,
            pltpu.SemaphoreType.DMA,
        ],
        compiler_params=_SC_PARAMS,
    )
    return f(h, xc8, row1d, col1d)


def _scatter_body(m_hbm, wd_hbm, row2d_hbm, zm_hbm, zw_hbm,
                  outm_hbm, outw_hbm,
                  accm, accw, idx, mv, wv):
    c = lax.axis_index("c")
    s = lax.axis_index("s")

    for q in range(NZ):
        r = s * RPT + q * ZCH
        pltpu.sync_copy(zm_hbm, accm.at[pl.ds(r, ZCH)])
        pltpu.sync_copy(zw_hbm, accw.at[pl.ds(r, ZCH)])
    plsc.subcore_barrier()

    def blk(q, carry):
        b = q * NS + s

        @pl.when(b < NBATCH)
        def _():
            base = b * SUBB
            pltpu.sync_copy(row2d_hbm.at[pl.ds(base, SUBB)], idx)
            for g in range(SUBB // LANES):
                sl = pl.ds(g * LANES, LANES)
                v = idx[sl]
                lv = v - c * HALF
                ok = (lv >= 0) & (lv < HALF)
                idx[sl] = jnp.where(ok, lv, HALF)
            pltpu.sync_copy(m_hbm.at[pl.ds(base, SUBB)], mv)
            pltpu.sync_copy(wd_hbm.at[pl.ds(base, SUBB)], wv)
            pltpu.sync_copy(mv, accm.at[idx], add=True)
            pltpu.sync_copy(wv, accw.at[idx], add=True)

        return carry

    lax.fori_loop(0, QMAX, blk, 0)
    plsc.subcore_barrier()

    for q in range(NZ):
        r = s * RPT + q * ZCH
        pltpu.sync_copy(accm.at[pl.ds(r, ZCH)], outm_hbm.at[pl.ds(c * AROWS + r, ZCH)])
        pltpu.sync_copy(accw.at[pl.ds(r, ZCH)], outw_hbm.at[pl.ds(c * AROWS + r, ZCH)])


def _scatter_call(m, wd4, row2d, zm, zw):
    mesh = plsc.VectorSubcoreMesh(core_axis_name="c", subcore_axis_name="s",
                                  num_cores=NC, num_subcores=NS)
    f = pl.kernel(
        _scatter_body,
        out_type=[jax.ShapeDtypeStruct((2 * AROWS, HID), jnp.float32),
                  jax.ShapeDtypeStruct((2 * AROWS, 8), jnp.float32)],
        mesh=mesh,
        scratch_types=[
            pltpu.VMEM_SHARED((AROWS, HID), jnp.float32),
            pltpu.VMEM_SHARED((AROWS, 8), jnp.float32),
            pltpu.VMEM((SUBB,), jnp.int32),
            pltpu.VMEM((SUBB, HID), jnp.float32),
            pltpu.VMEM((SUBB, 8), jnp.float32),
        ],
        compiler_params=_SC_PARAMS,
    )
    return f(m, wd4, row2d, zm, zw)


def _edge_body(hrow, hcol, xr8, xcv8, ea, wcat, b1, w2, b2, cw1, cb1, cw2p,
               m_o, wd_o):
    d3 = xr8[...][:, 0:3] - xcv8[...][:, 0:3]
    r2 = jnp.sum(d3 * d3, axis=1, keepdims=True)
    dr4 = jnp.concatenate([d3, r2], axis=1)
    bf = jnp.bfloat16
    ein = jnp.concatenate([hrow[...].astype(bf), hcol[...].astype(bf),
                           dr4.astype(bf), ea[...].astype(bf)], axis=1)
    z1 = jnp.dot(ein, wcat[...], preferred_element_type=jnp.float32) + b1[...][0:1]
    a = _silu(z1)
    m = _silu(jnp.dot(a.astype(bf), w2[...], preferred_element_type=jnp.float32) + b2[...][0:1])
    p = _silu(jnp.dot(m.astype(bf), cw1[...], preferred_element_type=jnp.float32) + cb1[...][0:1])
    cw8 = jnp.dot(p.astype(bf), cw2p[...], preferred_element_type=jnp.float32)
    m_o[...] = m
    lane8 = lax.broadcasted_iota(jnp.int32, (1, 8), 1)
    mul = (lane8 < 3).astype(jnp.float32)
    one = (lane8 == 3).astype(jnp.float32)
    dr8 = jnp.concatenate([dr4, jnp.zeros_like(dr4)], axis=1)
    wd_o[...] = dr8 * cw8[:, 0:1] * mul + one


def _edge_call(hrow, hcol, xr8, xcv8, ea, wcat, b1, w2, b2, cw1, cb1, cw2p):
    nblk = E // BE
    wspec = lambda shp: pl.BlockSpec(shp, lambda i: (0, 0))
    return pl.pallas_call(
        _edge_body,
        grid=(nblk,),
        in_specs=[
            pl.BlockSpec((BE, HID), lambda i: (i, 0)),
            pl.BlockSpec((BE, HID), lambda i: (i, 0)),
            pl.BlockSpec((BE, 8), lambda i: (i, 0)),
            pl.BlockSpec((BE, 8), lambda i: (i, 0)),
            pl.BlockSpec((BE, 4), lambda i: (i, 0)),
            wspec((72, HID)), wspec((8, HID)), wspec((HID, HID)),
            wspec((8, HID)), wspec((HID, HID)), wspec((8, HID)),
            wspec((HID, 8)),
        ],
        out_specs=[
            pl.BlockSpec((BE, HID), lambda i: (i, 0)),
            pl.BlockSpec((BE, 8), lambda i: (i, 0)),
        ],
        out_shape=[jax.ShapeDtypeStruct((E, HID), jnp.float32),
                   jax.ShapeDtypeStruct((E, 8), jnp.float32)],
    )(hrow, hcol, xr8, xcv8, ea, wcat, b1, w2, b2, cw1, cb1, cw2p)


def _node_body(h, xc8, vel8, outm, outw, cent8,
               vw1, vb1, vw2p, vb2p, nw1a, nw1b, nb1, nw2, nb2,
               ho, xco):
    hv = h[...]
    w = outw[...]
    cnt = jnp.maximum(w[:, 3:4], 1.0)
    lane8 = lax.broadcasted_iota(jnp.int32, (1, 8), 1)
    mul = (lane8 < 3).astype(jnp.float32)
    agg8 = w * mul / cnt
    xcv = xc8[...] + agg8
    vz = _silu(jnp.dot(hv, vw1[...], preferred_element_type=jnp.float32) + vb1[...][0:1])
    vw8 = jnp.dot(vz, vw2p[...], preferred_element_type=jnp.float32) + vb2p[...][0:1]
    xcv = xcv + vw8[:, 0:1] * vel8[...]
    xco[...] = xcv + cent8[...]
    magg = outm[...]
    z = _silu(jnp.dot(hv, nw1a[...], preferred_element_type=jnp.float32)
              + jnp.dot(magg, nw1b[...], preferred_element_type=jnp.float32)
              + nb1[...][0:1])
    ho[...] = jnp.dot(z, nw2[...], preferred_element_type=jnp.float32) + nb2[...][0:1]


def _node_call(h, xc8, vel8, outm, outw, cent8,
               vw1, vb1, vw2p, vb2p, nw1a, nw1b, nb1, nw2, nb2):
    nblk = N // NB
    wspec = lambda shp: pl.BlockSpec(shp, lambda i: (0, 0))
    return pl.pallas_call(
        _node_body,
        grid=(nblk,),
        in_specs=[
            pl.BlockSpec((NB, HID), lambda i: (i, 0)),
            pl.BlockSpec((NB, 8), lambda i: (i, 0)),
            pl.BlockSpec((NB, 8), lambda i: (i, 0)),
            pl.BlockSpec((NB, HID), lambda i: (i, 0)),
            pl.BlockSpec((NB, 8), lambda i: (i, 0)),
            pl.BlockSpec((NB, 8), lambda i: (i, 0)),
            wspec((HID, HID)), wspec((8, HID)), wspec((HID, 8)),
            wspec((8, 8)), wspec((HID, HID)), wspec((HID, HID)),
            wspec((8, HID)), wspec((HID, HID)), wspec((8, HID)),
        ],
        out_specs=[
            pl.BlockSpec((NB, HID), lambda i: (i, 0)),
            pl.BlockSpec((NB, 8), lambda i: (i, 0)),
        ],
        out_shape=[jax.ShapeDtypeStruct((N, HID), jnp.float32),
                   jax.ShapeDtypeStruct((N, 8), jnp.float32)],
    )(h, xc8, vel8, outm, outw, cent8,
      vw1, vb1, vw2p, vb2p, nw1a, nw1b, nb1, nw2, nb2)


def _pro_body(h8, x16, embw, embb, ic, cm, h0_o, xcc_o, cent_o):
    h0_o[...] = jnp.dot(h8[...], embw[...], preferred_element_type=jnp.float32) + embb[...][0:1]
    xv = x16[...]
    xcc_o[...] = jnp.dot(xv, ic[...], preferred_element_type=jnp.float32)
    cent_o[...] = jnp.dot(xv, cm[...], preferred_element_type=jnp.float32)


def _pro_call(h8, x16, embw, embb, ic, cm):
    nblk = 100
    wspec = lambda shp: pl.BlockSpec(shp, lambda i: (0, 0))
    return pl.pallas_call(
        _pro_body,
        grid=(nblk,),
        in_specs=[
            pl.BlockSpec((N // nblk, 8), lambda i: (i, 0)),
            pl.BlockSpec((N // G // nblk, 16), lambda i: (i, 0)),
            wspec((8, HID)), wspec((8, HID)), wspec((16, 16)), wspec((16, 16)),
        ],
        out_specs=[
            pl.BlockSpec((N // nblk, HID), lambda i: (i, 0)),
            pl.BlockSpec((N // G // nblk, 16), lambda i: (i, 0)),
            pl.BlockSpec((N // G // nblk, 16), lambda i: (i, 0)),
        ],
        out_shape=[jax.ShapeDtypeStruct((N, HID), jnp.float32),
                   jax.ShapeDtypeStruct((N // G, 16), jnp.float32),
                   jax.ShapeDtypeStruct((N // G, 16), jnp.float32)],
    )(h8, x16, embw, embb, ic, cm)


def _row8(b):
    out = jnp.zeros((8, b.shape[-1]), jnp.float32)
    return out.at[0].set(b)


def kernel(h, x, edges, vel, edge_attr, params):
    row = edges[0]
    col = edges[1]
    row2d = row

    h8 = jnp.pad(h, ((0, 0), (0, 8 - h.shape[1])))
    x16 = jnp.pad(x.reshape(N // G, 3 * G), ((0, 0), (0, 1)))
    vel8 = jnp.pad(vel, ((0, 0), (0, 5)))

    embw = jnp.pad(params['emb_W'], ((0, 8 - params['emb_W'].shape[0]), (0, 0)))
    embb = _row8(params['emb_b'])
    cmat = np.zeros((16, 16), np.float32)
    for i in range(3 * G):
        for j in range(3 * G):
            if i % 3 == j % 3:
                cmat[i, j] = 1.0 / G
    icmat = np.eye(16, dtype=np.float32)
    icmat[15, 15] = 0.0
    icmat = icmat - cmat
    cmat = jnp.asarray(cmat)
    icmat = jnp.asarray(icmat)

    H, xcc16, cent16 = _pro_call(h8, x16, embw, embb, icmat, cmat)
    xc8 = jnp.pad(xcc16[:, :3 * G].reshape(N, 3), ((0, 0), (0, 5)))
    cent8 = jnp.pad(cent16[:, :3 * G].reshape(N, 3), ((0, 0), (0, 5)))
    zcent8 = jnp.zeros_like(cent8)

    zm = jnp.zeros((ZCH, HID), jnp.float32)
    zw = jnp.zeros((ZCH, 8), jnp.float32)

    for li, lp in enumerate(params['layers']):
        wcat = jnp.concatenate([
            lp['edge_W1'][0:2 * HID],
            jnp.zeros((3, HID), jnp.float32),
            lp['edge_W1'][2 * HID:2 * HID + 1],
            lp['edge_W1'][2 * HID + 1:],
        ], axis=0)
        cw2p = jnp.pad(lp['coord_W2'], ((0, 0), (0, 7)))
        vw2p = jnp.pad(lp['vel_W2'], ((0, 0), (0, 7)))
        vb2p = jnp.zeros((8, 8), jnp.float32).at[0, 0].set(lp['vel_b2'][0])
        nw1a = lp['node_W1'][0:HID]
        nw1b = lp['node_W1'][HID:]

        hrow, hcol, xr8, xcv8 = _gather_call(H, xc8, row, col)
        m, wd4 = _edge_call(hrow, hcol, xr8, xcv8, edge_attr,
                            wcat.astype(jnp.bfloat16),
                            _row8(lp['edge_b1']),
                            lp['edge_W2'].astype(jnp.bfloat16),
                            _row8(lp['edge_b2']),
                            lp['coord_W1'].astype(jnp.bfloat16),
                            _row8(lp['coord_b1']),
                            cw2p.astype(jnp.bfloat16))
        outm, outw = _scatter_call(m, wd4, row2d, zm, zw)
        outm = jnp.concatenate([outm[:HALF], outm[AROWS:AROWS + HALF]], axis=0)
        outw = jnp.concatenate([outw[:HALF], outw[AROWS:AROWS + HALF]], axis=0)
        H, xc8 = _node_call(H, xc8, vel8, outm, outw,
                            cent8 if li == len(params['layers']) - 1 else zcent8,
                            lp['vel_W1'], _row8(lp['vel_b1']), vw2p, vb2p,
                            nw1a, nw1b, _row8(lp['node_b1']),
                            lp['node_W2'], _row8(lp['node_b2']))

    return xc8[:, :3]


# speedup vs baseline: 1.2281x; 1.2281x over previous
import functools

import jax
import jax.numpy as jnp
import numpy as np
from jax import lax
from jax.experimental import pallas as pl
from jax.experimental.pallas import tpu as pltpu
from jax.experimental.pallas import tpu_sc as plsc

N = 100000
E = 1600000
HID = 32
G = 5
HALF = N // 2
AROWS = 50016
NC, NS, LANES = 2, 16, 16
NW = NC * NS

BLK = 400
SUB = 80
NSUB = BLK // SUB
EPT_G = E // NW
NBLK_G = EPT_G // BLK
SUBB = 128
NBATCH = E // SUBB
QMAX = -(-NBATCH // NS)
ZCH = 521
RPT = AROWS // NS
NZ = RPT // ZCH

BE = 5000
NB = 400

_SC_PARAMS = pltpu.CompilerParams(use_tc_tiling_on_sc=False)


def _silu(v):
    return v * jax.nn.sigmoid(v)


def _gather_body(h_hbm, xc_hbm, row_hbm, col_hbm,
                 hrow_hbm, hcol_hbm, xr_hbm, xcv_hbm,
                 idxr, idxc, hrow_v, hcol_v, xr_v, xcv_v,
                 sem0, sem1, sem2, sem3):
    c = lax.axis_index("c")
    s = lax.axis_index("s")
    wid = s * NC + c

    def blk(j, carry):
        base = wid * EPT_G + j * BLK
        pltpu.sync_copy(row_hbm.at[pl.ds(base, BLK)], idxr)
        pltpu.sync_copy(col_hbm.at[pl.ds(base, BLK)], idxc)
        cps = []
        for t in range(NSUB):
            sl = pl.ds(t * SUB, SUB)
            cps.append(pltpu.async_copy(h_hbm.at[idxr.at[sl]], hrow_v.at[sl], sem0))
            cps.append(pltpu.async_copy(h_hbm.at[idxc.at[sl]], hcol_v.at[sl], sem1))
            cps.append(pltpu.async_copy(xc_hbm.at[idxr.at[sl]], xr_v.at[sl], sem2))
            cps.append(pltpu.async_copy(xc_hbm.at[idxc.at[sl]], xcv_v.at[sl], sem3))
        for cp in cps:
            cp.wait()
        pltpu.sync_copy(hrow_v, hrow_hbm.at[pl.ds(base, BLK)])
        pltpu.sync_copy(hcol_v, hcol_hbm.at[pl.ds(base, BLK)])
        pltpu.sync_copy(xr_v, xr_hbm.at[pl.ds(base, BLK)])
        pltpu.sync_copy(xcv_v, xcv_hbm.at[pl.ds(base, BLK)])
        return carry

    lax.fori_loop(0, NBLK_G, blk, 0)


def _gather_call(h, xc8, row1d, col1d):
    mesh = plsc.VectorSubcoreMesh(core_axis_name="c", subcore_axis_name="s",
                                  num_cores=NC, num_subcores=NS)
    f = pl.kernel(
        _gather_body,
        out_type=[jax.ShapeDtypeStruct((E, HID), jnp.float32),
                  jax.ShapeDtypeStruct((E, HID), jnp.float32),
                  jax.ShapeDtypeStruct((E, 8), jnp.float32),
                  jax.ShapeDtypeStruct((E, 8), jnp.float32)],
        mesh=mesh,
        scratch_types=[
            pltpu.VMEM((BLK,), jnp.int32),
            pltpu.VMEM((BLK,), jnp.int32),
            pltpu.VMEM((BLK, HID), jnp.float32),
            pltpu.VMEM((BLK, HID), jnp.float32),
            pltpu.VMEM((BLK, 8), jnp.float32),
            pltpu.VMEM((BLK, 8), jnp.float32),
            pltpu.SemaphoreType.DMA,
            pltpu.SemaphoreType.DMA,
            pltpu.SemaphoreType.DMA,
            pltpu.SemaphoreType.DMA,
        ],
        compiler_params=_SC_PARAMS,
    )
    return f(h, xc8, row1d, col1d)


def _scatter_body(m_hbm, wd_hbm, row2d_hbm, zm_hbm, zw_hbm,
                  outm_hbm, outw_hbm,
                  accm, accw, idx, mv, wv):
    c = lax.axis_index("c")
    s = lax.axis_index("s")

    for q in range(NZ):
        r = s * RPT + q * ZCH
        pltpu.sync_copy(zm_hbm, accm.at[pl.ds(r, ZCH)])
        pltpu.sync_copy(zw_hbm, accw.at[pl.ds(r, ZCH)])
    plsc.subcore_barrier()

    def blk(q, carry):
        b = q * NS + s

        @pl.when(b < NBATCH)
        def _():
            base = b * SUBB
            pltpu.sync_copy(row2d_hbm.at[pl.ds(base, SUBB)], idx)
            for g in range(SUBB // LANES):
                sl = pl.ds(g * LANES, LANES)
                v = idx[sl]
                lv = v - c * HALF
                ok = (lv >= 0) & (lv < HALF)
                idx[sl] = jnp.where(ok, lv, HALF)
            pltpu.sync_copy(m_hbm.at[pl.ds(base, SUBB)], mv)
            pltpu.sync_copy(wd_hbm.at[pl.ds(base, SUBB)], wv)
            pltpu.sync_copy(mv, accm.at[idx], add=True)
            pltpu.sync_copy(wv, accw.at[idx], add=True)

        return carry

    lax.fori_loop(0, QMAX, blk, 0)
    plsc.subcore_barrier()

    for q in range(NZ):
        r = s * RPT + q * ZCH
        pltpu.sync_copy(accm.at[pl.ds(r, ZCH)], outm_hbm.at[pl.ds(c * AROWS + r, ZCH)])
        pltpu.sync_copy(accw.at[pl.ds(r, ZCH)], outw_hbm.at[pl.ds(c * AROWS + r, ZCH)])


def _scatter_call(m, wd4, row2d, zm, zw):
    mesh = plsc.VectorSubcoreMesh(core_axis_name="c", subcore_axis_name="s",
                                  num_cores=NC, num_subcores=NS)
    f = pl.kernel(
        _scatter_body,
        out_type=[jax.ShapeDtypeStruct((2 * AROWS, HID), jnp.float32),
                  jax.ShapeDtypeStruct((2 * AROWS, 8), jnp.float32)],
        mesh=mesh,
        scratch_types=[
            pltpu.VMEM_SHARED((AROWS, HID), jnp.float32),
            pltpu.VMEM_SHARED((AROWS, 8), jnp.float32),
            pltpu.VMEM((SUBB,), jnp.int32),
            pltpu.VMEM((SUBB, HID), jnp.float32),
            pltpu.VMEM((SUBB, 8), jnp.float32),
        ],
        compiler_params=_SC_PARAMS,
    )
    return f(m, wd4, row2d, zm, zw)


def _edge_body(hrow, hcol, xr8, xcv8, ea, wcat, b1, w2, b2, cw1, cb1, cw2p,
               m_o, wd_o):
    d3 = xr8[...][:, 0:3] - xcv8[...][:, 0:3]
    r2 = jnp.sum(d3 * d3, axis=1, keepdims=True)
    dr4 = jnp.concatenate([d3, r2], axis=1)
    ein = jnp.concatenate([hrow[...], hcol[...], dr4, ea[...]], axis=1)
    z1 = jnp.dot(ein, wcat[...], preferred_element_type=jnp.float32) + b1[...][0:1]
    a = _silu(z1)
    m = _silu(jnp.dot(a, w2[...], preferred_element_type=jnp.float32) + b2[...][0:1])
    p = _silu(jnp.dot(m, cw1[...], preferred_element_type=jnp.float32) + cb1[...][0:1])
    cw8 = jnp.dot(p, cw2p[...], preferred_element_type=jnp.float32)
    m_o[...] = m
    lane8 = lax.broadcasted_iota(jnp.int32, (1, 8), 1)
    mul = (lane8 < 3).astype(jnp.float32)
    one = (lane8 == 3).astype(jnp.float32)
    dr8 = jnp.concatenate([dr4, jnp.zeros_like(dr4)], axis=1)
    wd_o[...] = dr8 * cw8[:, 0:1] * mul + one


def _edge_call(hrow, hcol, xr8, xcv8, ea, wcat, b1, w2, b2, cw1, cb1, cw2p):
    nblk = E // BE
    wspec = lambda shp: pl.BlockSpec(shp, lambda i: (0, 0))
    return pl.pallas_call(
        _edge_body,
        grid=(nblk,),
        in_specs=[
            pl.BlockSpec((BE, HID), lambda i: (i, 0)),
            pl.BlockSpec((BE, HID), lambda i: (i, 0)),
            pl.BlockSpec((BE, 8), lambda i: (i, 0)),
            pl.BlockSpec((BE, 8), lambda i: (i, 0)),
            pl.BlockSpec((BE, 4), lambda i: (i, 0)),
            wspec((72, HID)), wspec((8, HID)), wspec((HID, HID)),
            wspec((8, HID)), wspec((HID, HID)), wspec((8, HID)),
            wspec((HID, 8)),
        ],
        out_specs=[
            pl.BlockSpec((BE, HID), lambda i: (i, 0)),
            pl.BlockSpec((BE, 8), lambda i: (i, 0)),
        ],
        out_shape=[jax.ShapeDtypeStruct((E, HID), jnp.float32),
                   jax.ShapeDtypeStruct((E, 8), jnp.float32)],
    )(hrow, hcol, xr8, xcv8, ea, wcat, b1, w2, b2, cw1, cb1, cw2p)


def _node_body(h, xc8, vel8, outm, outw, cent8,
               vw1, vb1, vw2p, vb2p, nw1a, nw1b, nb1, nw2, nb2,
               ho, xco):
    hv = h[...]
    w = outw[...]
    cnt = jnp.maximum(w[:, 3:4], 1.0)
    lane8 = lax.broadcasted_iota(jnp.int32, (1, 8), 1)
    mul = (lane8 < 3).astype(jnp.float32)
    agg8 = w * mul / cnt
    xcv = xc8[...] + agg8
    vz = _silu(jnp.dot(hv, vw1[...], preferred_element_type=jnp.float32) + vb1[...][0:1])
    vw8 = jnp.dot(vz, vw2p[...], preferred_element_type=jnp.float32) + vb2p[...][0:1]
    xcv = xcv + vw8[:, 0:1] * vel8[...]
    xco[...] = xcv + cent8[...]
    magg = outm[...]
    z = _silu(jnp.dot(hv, nw1a[...], preferred_element_type=jnp.float32)
              + jnp.dot(magg, nw1b[...], preferred_element_type=jnp.float32)
              + nb1[...][0:1])
    ho[...] = jnp.dot(z, nw2[...], preferred_element_type=jnp.float32) + nb2[...][0:1]


def _node_call(h, xc8, vel8, outm, outw, cent8,
               vw1, vb1, vw2p, vb2p, nw1a, nw1b, nb1, nw2, nb2):
    nblk = N // NB
    wspec = lambda shp: pl.BlockSpec(shp, lambda i: (0, 0))
    return pl.pallas_call(
        _node_body,
        grid=(nblk,),
        in_specs=[
            pl.BlockSpec((NB, HID), lambda i: (i, 0)),
            pl.BlockSpec((NB, 8), lambda i: (i, 0)),
            pl.BlockSpec((NB, 8), lambda i: (i, 0)),
            pl.BlockSpec((NB, HID), lambda i: (i, 0)),
            pl.BlockSpec((NB, 8), lambda i: (i, 0)),
            pl.BlockSpec((NB, 8), lambda i: (i, 0)),
            wspec((HID, HID)), wspec((8, HID)), wspec((HID, 8)),
            wspec((8, 8)), wspec((HID, HID)), wspec((HID, HID)),
            wspec((8, HID)), wspec((HID, HID)), wspec((8, HID)),
        ],
        out_specs=[
            pl.BlockSpec((NB, HID), lambda i: (i, 0)),
            pl.BlockSpec((NB, 8), lambda i: (i, 0)),
        ],
        out_shape=[jax.ShapeDtypeStruct((N, HID), jnp.float32),
                   jax.ShapeDtypeStruct((N, 8), jnp.float32)],
    )(h, xc8, vel8, outm, outw, cent8,
      vw1, vb1, vw2p, vb2p, nw1a, nw1b, nb1, nw2, nb2)


def _pro_body(h8, x16, embw, embb, ic, cm, h0_o, xcc_o, cent_o):
    h0_o[...] = jnp.dot(h8[...], embw[...], preferred_element_type=jnp.float32) + embb[...][0:1]
    xv = x16[...]
    xcc_o[...] = jnp.dot(xv, ic[...], preferred_element_type=jnp.float32)
    cent_o[...] = jnp.dot(xv, cm[...], preferred_element_type=jnp.float32)


def _pro_call(h8, x16, embw, embb, ic, cm):
    nblk = 100
    wspec = lambda shp: pl.BlockSpec(shp, lambda i: (0, 0))
    return pl.pallas_call(
        _pro_body,
        grid=(nblk,),
        in_specs=[
            pl.BlockSpec((N // nblk, 8), lambda i: (i, 0)),
            pl.BlockSpec((N // G // nblk, 16), lambda i: (i, 0)),
            wspec((8, HID)), wspec((8, HID)), wspec((16, 16)), wspec((16, 16)),
        ],
        out_specs=[
            pl.BlockSpec((N // nblk, HID), lambda i: (i, 0)),
            pl.BlockSpec((N // G // nblk, 16), lambda i: (i, 0)),
            pl.BlockSpec((N // G // nblk, 16), lambda i: (i, 0)),
        ],
        out_shape=[jax.ShapeDtypeStruct((N, HID), jnp.float32),
                   jax.ShapeDtypeStruct((N // G, 16), jnp.float32),
                   jax.ShapeDtypeStruct((N // G, 16), jnp.float32)],
    )(h8, x16, embw, embb, ic, cm)


def _row8(b):
    out = jnp.zeros((8, b.shape[-1]), jnp.float32)
    return out.at[0].set(b)


def kernel(h, x, edges, vel, edge_attr, params):
    row = edges[0]
    col = edges[1]
    row2d = row

    h8 = jnp.pad(h, ((0, 0), (0, 8 - h.shape[1])))
    x16 = jnp.pad(x.reshape(N // G, 3 * G), ((0, 0), (0, 1)))
    vel8 = jnp.pad(vel, ((0, 0), (0, 5)))

    embw = jnp.pad(params['emb_W'], ((0, 8 - params['emb_W'].shape[0]), (0, 0)))
    embb = _row8(params['emb_b'])
    cmat = np.zeros((16, 16), np.float32)
    for i in range(3 * G):
        for j in range(3 * G):
            if i % 3 == j % 3:
                cmat[i, j] = 1.0 / G
    icmat = np.eye(16, dtype=np.float32)
    icmat[15, 15] = 0.0
    icmat = icmat - cmat
    cmat = jnp.asarray(cmat)
    icmat = jnp.asarray(icmat)

    H, xcc16, cent16 = _pro_call(h8, x16, embw, embb, icmat, cmat)
    xc8 = jnp.pad(xcc16[:, :3 * G].reshape(N, 3), ((0, 0), (0, 5)))
    cent8 = jnp.pad(cent16[:, :3 * G].reshape(N, 3), ((0, 0), (0, 5)))
    zcent8 = jnp.zeros_like(cent8)

    zm = jnp.zeros((ZCH, HID), jnp.float32)
    zw = jnp.zeros((ZCH, 8), jnp.float32)

    for li, lp in enumerate(params['layers']):
        wcat = jnp.concatenate([
            lp['edge_W1'][0:2 * HID],
            jnp.zeros((3, HID), jnp.float32),
            lp['edge_W1'][2 * HID:2 * HID + 1],
            lp['edge_W1'][2 * HID + 1:],
        ], axis=0)
        cw2p = jnp.pad(lp['coord_W2'], ((0, 0), (0, 7)))
        vw2p = jnp.pad(lp['vel_W2'], ((0, 0), (0, 7)))
        vb2p = jnp.zeros((8, 8), jnp.float32).at[0, 0].set(lp['vel_b2'][0])
        nw1a = lp['node_W1'][0:HID]
        nw1b = lp['node_W1'][HID:]

        hrow, hcol, xr8, xcv8 = _gather_call(H, xc8, row, col)
        m, wd4 = _edge_call(hrow, hcol, xr8, xcv8, edge_attr,
                            wcat, _row8(lp['edge_b1']), lp['edge_W2'],
                            _row8(lp['edge_b2']), lp['coord_W1'],
                            _row8(lp['coord_b1']), cw2p)
        outm, outw = _scatter_call(m, wd4, row2d, zm, zw)
        outm = jnp.concatenate([outm[:HALF], outm[AROWS:AROWS + HALF]], axis=0)
        outw = jnp.concatenate([outw[:HALF], outw[AROWS:AROWS + HALF]], axis=0)
        H, xc8 = _node_call(H, xc8, vel8, outm, outw,
                            cent8 if li == len(params['layers']) - 1 else zcent8,
                            lp['vel_W1'], _row8(lp['vel_b1']), vw2p, vb2p,
                            nw1a, nw1b, _row8(lp['node_b1']),
                            lp['node_W2'], _row8(lp['node_b2']))

    return xc8[:, :3]


# speedup vs baseline: 2.1337x; 1.7374x over previous
import functools

import jax
import jax.numpy as jnp
import numpy as np
from jax import lax
from jax.experimental import pallas as pl
from jax.experimental.pallas import tpu as pltpu
from jax.experimental.pallas import tpu_sc as plsc

N = 100000
E = 1600000
HID = 32
G = 5
HALF = N // 2
AROWS = 50016
NC, NS, LANES = 2, 16, 16
NW = NC * NS

BLK = 400
SUB = 80
NSUB = BLK // SUB
EPT_G = E // NW
NBLK_G = EPT_G // BLK
SUBB = 128
NBATCH = E // SUBB
QMAX = -(-NBATCH // NS)
ZCH = 521
RPT = AROWS // NS
NZ = RPT // ZCH

BE = 6400
NB = 400

_SC_PARAMS = pltpu.CompilerParams(use_tc_tiling_on_sc=False)


def _silu(v):
    return v * jax.nn.sigmoid(v)


def _gather_body(h_hbm, xc_hbm, row_hbm, col_hbm, ea_hbm, zp_hbm,
                 hrow_hbm, hcol_hbm, rec_hbm,
                 idxr, idxc, hrow_v, hcol_v, xr_v, xcv_v, eaz_v,
                 sem0, sem1, sem2, sem3):
    c = lax.axis_index("c")
    s = lax.axis_index("s")
    wid = s * NC + c
    pltpu.sync_copy(zp_hbm, eaz_v)

    def blk(j, carry):
        base = wid * EPT_G + j * BLK
        pltpu.sync_copy(row_hbm.at[pl.ds(base, BLK)], idxr)
        pltpu.sync_copy(col_hbm.at[pl.ds(base, BLK)], idxc)
        pltpu.sync_copy(ea_hbm.at[pl.ds(base, BLK)], eaz_v.at[pl.ds(0, BLK), pl.ds(0, 4)])
        cps = []
        for t in range(NSUB):
            sl = pl.ds(t * SUB, SUB)
            cps.append(pltpu.async_copy(h_hbm.at[idxr.at[sl]], hrow_v.at[sl], sem0))
            cps.append(pltpu.async_copy(h_hbm.at[idxc.at[sl]], hcol_v.at[sl], sem1))
            cps.append(pltpu.async_copy(xc_hbm.at[idxr.at[sl]], xr_v.at[sl], sem2))
            cps.append(pltpu.async_copy(xc_hbm.at[idxc.at[sl]], xcv_v.at[sl], sem3))
        for cp in cps:
            cp.wait()
        pltpu.sync_copy(hrow_v, hrow_hbm.at[pl.ds(base, BLK)])
        pltpu.sync_copy(hcol_v, hcol_hbm.at[pl.ds(base, BLK)])
        pltpu.sync_copy(xr_v, rec_hbm.at[pl.ds(base, BLK), pl.ds(0, 8)])
        pltpu.sync_copy(xcv_v, rec_hbm.at[pl.ds(base, BLK), pl.ds(8, 8)])
        pltpu.sync_copy(eaz_v, rec_hbm.at[pl.ds(base, BLK), pl.ds(16, 16)])
        return carry

    lax.fori_loop(0, NBLK_G, blk, 0)


def _gather_call(h, xc8, row1d, col1d, ea, zp16):
    mesh = plsc.VectorSubcoreMesh(core_axis_name="c", subcore_axis_name="s",
                                  num_cores=NC, num_subcores=NS)
    f = pl.kernel(
        _gather_body,
        out_type=[jax.ShapeDtypeStruct((E, HID), jnp.float32),
                  jax.ShapeDtypeStruct((E, HID), jnp.float32),
                  jax.ShapeDtypeStruct((E, HID), jnp.float32)],
        mesh=mesh,
        scratch_types=[
            pltpu.VMEM((BLK,), jnp.int32),
            pltpu.VMEM((BLK,), jnp.int32),
            pltpu.VMEM((BLK, HID), jnp.float32),
            pltpu.VMEM((BLK, HID), jnp.float32),
            pltpu.VMEM((BLK, 8), jnp.float32),
            pltpu.VMEM((BLK, 8), jnp.float32),
            pltpu.VMEM((BLK, 16), jnp.float32),
            pltpu.SemaphoreType.DMA,
            pltpu.SemaphoreType.DMA,
            pltpu.SemaphoreType.DMA,
            pltpu.SemaphoreType.DMA,
        ],
        compiler_params=_SC_PARAMS,
    )
    return f(h, xc8, row1d, col1d, ea, zp16)


def _scatter_body(m_hbm, wd_hbm, row2d_hbm, zm_hbm, zw_hbm,
                  outm_hbm, outw_hbm,
                  accm, accw, idx, mv, wv):
    c = lax.axis_index("c")
    s = lax.axis_index("s")

    for q in range(NZ):
        r = s * RPT + q * ZCH
        pltpu.sync_copy(zm_hbm, accm.at[pl.ds(r, ZCH)])
        pltpu.sync_copy(zw_hbm, accw.at[pl.ds(r, ZCH)])
    plsc.subcore_barrier()

    def blk(q, carry):
        b = q * NS + s

        @pl.when(b < NBATCH)
        def _():
            base = b * SUBB
            pltpu.sync_copy(row2d_hbm.at[pl.ds(base, SUBB)], idx)
            for g in range(SUBB // LANES):
                sl = pl.ds(g * LANES, LANES)
                v = idx[sl]
                lv = v - c * HALF
                ok = (lv >= 0) & (lv < HALF)
                idx[sl] = jnp.where(ok, lv, HALF)
            pltpu.sync_copy(m_hbm.at[pl.ds(base, SUBB)], mv)
            pltpu.sync_copy(wd_hbm.at[pl.ds(base, SUBB), pl.ds(0, 8)], wv)
            pltpu.sync_copy(mv, accm.at[idx], add=True)
            pltpu.sync_copy(wv, accw.at[idx], add=True)

        return carry

    lax.fori_loop(0, QMAX, blk, 0)
    plsc.subcore_barrier()

    for q in range(NZ):
        r = s * RPT + q * ZCH
        pltpu.sync_copy(accm.at[pl.ds(r, ZCH)], outm_hbm.at[pl.ds(c * AROWS + r, ZCH)])
        pltpu.sync_copy(accw.at[pl.ds(r, ZCH)], outw_hbm.at[pl.ds(c * AROWS + r, ZCH)])


def _scatter_call(m, wd4, row2d, zm, zw):
    mesh = plsc.VectorSubcoreMesh(core_axis_name="c", subcore_axis_name="s",
                                  num_cores=NC, num_subcores=NS)
    f = pl.kernel(
        _scatter_body,
        out_type=[jax.ShapeDtypeStruct((2 * AROWS, HID), jnp.float32),
                  jax.ShapeDtypeStruct((2 * AROWS, 8), jnp.float32)],
        mesh=mesh,
        scratch_types=[
            pltpu.VMEM_SHARED((AROWS, HID), jnp.float32),
            pltpu.VMEM_SHARED((AROWS, 8), jnp.float32),
            pltpu.VMEM((SUBB,), jnp.int32),
            pltpu.VMEM((SUBB, HID), jnp.float32),
            pltpu.VMEM((SUBB, 8), jnp.float32),
        ],
        compiler_params=_SC_PARAMS,
    )
    return f(m, wd4, row2d, zm, zw)


def _edge_body(hrowP, hcolP, recP,
               dm, bdw1a, bdw1b, wr, wea, b1t, bdw2, b2t, bdcw1, cb1t,
               bdcw2, sp, m_o, wd_o):
    f32 = jnp.float32
    dot = lambda a, b: jnp.dot(a, b, preferred_element_type=f32)
    rec = recP[...]
    d = dot(rec, dm[...])
    dsq = d * d
    z1 = (dot(hrowP[...], bdw1a[...]) + dot(hcolP[...], bdw1b[...])
          + dot(dsq, wr[...]) + dot(rec, wea[...]) + b1t[...][0:1])
    a = _silu(z1)
    m = _silu(dot(a, bdw2[...]) + b2t[...][0:1])
    p = _silu(dot(m, bdcw1[...]) + cb1t[...][0:1])
    cw4 = dot(p, bdcw2[...])
    m_o[...] = m
    cwsp = dot(cw4, sp[...])
    l128 = lax.broadcasted_iota(jnp.int32, (1, 128), 1)
    wd_o[...] = d * cwsp + ((l128 % 32) == 3).astype(f32)


def _edge_call(hrowP, hcolP, recP,
               dm, bdw1a, bdw1b, wr, wea, b1t, bdw2, b2t, bdcw1, cb1t,
               bdcw2, sp):
    nblk = E // BE
    wspec = lambda shp: pl.BlockSpec(shp, lambda i: (0, 0))
    bigspec = pl.BlockSpec((BE // 4, 128), lambda i: (i, 0))
    return pl.pallas_call(
        _edge_body,
        grid=(nblk,),
        in_specs=[
            bigspec, bigspec, bigspec,
            wspec((128, 128)), wspec((128, 128)), wspec((128, 128)),
            wspec((128, 128)), wspec((128, 128)), wspec((8, 128)),
            wspec((128, 128)), wspec((8, 128)), wspec((128, 128)),
            wspec((8, 128)), wspec((128, 128)), wspec((128, 128)),
        ],
        out_specs=[bigspec, bigspec],
        out_shape=[jax.ShapeDtypeStruct((E // 4, 128), jnp.float32),
                   jax.ShapeDtypeStruct((E // 4, 128), jnp.float32)],
    )(hrowP, hcolP, recP,
      dm, bdw1a, bdw1b, wr, wea, b1t, bdw2, b2t, bdcw1, cb1t, bdcw2, sp)


def _node_body(h, xc8, vel8, outm, outw, cent8,
               vw1, vb1, vw2p, vb2p, nw1a, nw1b, nb1, nw2, nb2,
               ho, xco):
    hv = h[...]
    w = outw[...]
    cnt = jnp.maximum(w[:, 3:4], 1.0)
    lane8 = lax.broadcasted_iota(jnp.int32, (1, 8), 1)
    mul = (lane8 < 3).astype(jnp.float32)
    agg8 = w * mul / cnt
    xcv = xc8[...] + agg8
    vz = _silu(jnp.dot(hv, vw1[...], preferred_element_type=jnp.float32) + vb1[...][0:1])
    vw8 = jnp.dot(vz, vw2p[...], preferred_element_type=jnp.float32) + vb2p[...][0:1]
    xcv = xcv + vw8[:, 0:1] * vel8[...]
    xco[...] = xcv + cent8[...]
    magg = outm[...]
    z = _silu(jnp.dot(hv, nw1a[...], preferred_element_type=jnp.float32)
              + jnp.dot(magg, nw1b[...], preferred_element_type=jnp.float32)
              + nb1[...][0:1])
    ho[...] = jnp.dot(z, nw2[...], preferred_element_type=jnp.float32) + nb2[...][0:1]


def _node_call(h, xc8, vel8, outm, outw, cent8,
               vw1, vb1, vw2p, vb2p, nw1a, nw1b, nb1, nw2, nb2):
    nblk = N // NB
    wspec = lambda shp: pl.BlockSpec(shp, lambda i: (0, 0))
    return pl.pallas_call(
        _node_body,
        grid=(nblk,),
        in_specs=[
            pl.BlockSpec((NB, HID), lambda i: (i, 0)),
            pl.BlockSpec((NB, 8), lambda i: (i, 0)),
            pl.BlockSpec((NB, 8), lambda i: (i, 0)),
            pl.BlockSpec((NB, HID), lambda i: (i, 0)),
            pl.BlockSpec((NB, 8), lambda i: (i, 0)),
            pl.BlockSpec((NB, 8), lambda i: (i, 0)),
            wspec((HID, HID)), wspec((8, HID)), wspec((HID, 8)),
            wspec((8, 8)), wspec((HID, HID)), wspec((HID, HID)),
            wspec((8, HID)), wspec((HID, HID)), wspec((8, HID)),
        ],
        out_specs=[
            pl.BlockSpec((NB, HID), lambda i: (i, 0)),
            pl.BlockSpec((NB, 8), lambda i: (i, 0)),
        ],
        out_shape=[jax.ShapeDtypeStruct((N, HID), jnp.float32),
                   jax.ShapeDtypeStruct((N, 8), jnp.float32)],
    )(h, xc8, vel8, outm, outw, cent8,
      vw1, vb1, vw2p, vb2p, nw1a, nw1b, nb1, nw2, nb2)


def _pro_body(h8, x16, embw, embb, ic, cm, h0_o, xcc_o, cent_o):
    h0_o[...] = jnp.dot(h8[...], embw[...], preferred_element_type=jnp.float32) + embb[...][0:1]
    xv = x16[...]
    xcc_o[...] = jnp.dot(xv, ic[...], preferred_element_type=jnp.float32)
    cent_o[...] = jnp.dot(xv, cm[...], preferred_element_type=jnp.float32)


def _pro_call(h8, x16, embw, embb, ic, cm):
    nblk = 100
    wspec = lambda shp: pl.BlockSpec(shp, lambda i: (0, 0))
    return pl.pallas_call(
        _pro_body,
        grid=(nblk,),
        in_specs=[
            pl.BlockSpec((N // nblk, 8), lambda i: (i, 0)),
            pl.BlockSpec((N // G // nblk, 16), lambda i: (i, 0)),
            wspec((8, HID)), wspec((8, HID)), wspec((16, 16)), wspec((16, 16)),
        ],
        out_specs=[
            pl.BlockSpec((N // nblk, HID), lambda i: (i, 0)),
            pl.BlockSpec((N // G // nblk, 16), lambda i: (i, 0)),
            pl.BlockSpec((N // G // nblk, 16), lambda i: (i, 0)),
        ],
        out_shape=[jax.ShapeDtypeStruct((N, HID), jnp.float32),
                   jax.ShapeDtypeStruct((N // G, 16), jnp.float32),
                   jax.ShapeDtypeStruct((N // G, 16), jnp.float32)],
    )(h8, x16, embw, embb, ic, cm)


def _row8(b):
    out = jnp.zeros((8, b.shape[-1]), jnp.float32)
    return out.at[0].set(b)


def kernel(h, x, edges, vel, edge_attr, params):
    row = edges[0]
    col = edges[1]
    row2d = row

    h8 = jnp.pad(h, ((0, 0), (0, 8 - h.shape[1])))
    x16 = jnp.pad(x.reshape(N // G, 3 * G), ((0, 0), (0, 1)))
    vel8 = jnp.pad(vel, ((0, 0), (0, 5)))

    embw = jnp.pad(params['emb_W'], ((0, 8 - params['emb_W'].shape[0]), (0, 0)))
    embb = _row8(params['emb_b'])
    cmat = np.zeros((16, 16), np.float32)
    for i in range(3 * G):
        for j in range(3 * G):
            if i % 3 == j % 3:
                cmat[i, j] = 1.0 / G
    icmat = np.eye(16, dtype=np.float32)
    icmat[15, 15] = 0.0
    icmat = icmat - cmat
    cmat = jnp.asarray(cmat)
    icmat = jnp.asarray(icmat)

    H, xcc16, cent16 = _pro_call(h8, x16, embw, embb, icmat, cmat)
    xc8 = jnp.pad(xcc16[:, :3 * G].reshape(N, 3), ((0, 0), (0, 5)))
    cent8 = jnp.pad(cent16[:, :3 * G].reshape(N, 3), ((0, 0), (0, 5)))
    zcent8 = jnp.zeros_like(cent8)

    zm = jnp.zeros((ZCH, HID), jnp.float32)
    zw = jnp.zeros((ZCH, 8), jnp.float32)

    i4 = jnp.eye(4, dtype=jnp.float32)
    m3c32 = jnp.asarray((np.arange(32) < 3).astype(np.float32).reshape(32, 1))
    dmnp = np.zeros((128, 128), np.float32)
    for g in range(4):
        for k in range(3):
            dmnp[32 * g + k, 32 * g + k] = 1.0
            dmnp[32 * g + k + 8, 32 * g + k] = -1.0
    dm = jnp.asarray(dmnp)
    b3 = jnp.zeros((32, 32), jnp.float32).at[0, 0:3].set(1.0)
    sp = jnp.kron(i4, b3)
    zp16 = jnp.zeros((BLK, 16), jnp.float32)
    _r8w = lambda b: jnp.zeros((8, 128), jnp.float32).at[0].set(jnp.tile(b, 4))

    for li, lp in enumerate(params['layers']):
        bdw1a = jnp.kron(i4, lp['edge_W1'][0:HID])
        bdw1b = jnp.kron(i4, lp['edge_W1'][HID:2 * HID])
        wr = jnp.kron(i4, m3c32 @ lp['edge_W1'][2 * HID:2 * HID + 1])
        wea = jnp.kron(i4, jnp.zeros((HID, HID), jnp.float32
                                     ).at[16:20].set(lp['edge_W1'][2 * HID + 1:]))
        bdw2 = jnp.kron(i4, lp['edge_W2'])
        bdcw1 = jnp.kron(i4, lp['coord_W1'])
        bdcw2 = jnp.kron(i4, jnp.pad(lp['coord_W2'], ((0, 0), (0, 31))))
        vw2p = jnp.pad(lp['vel_W2'], ((0, 0), (0, 7)))
        vb2p = jnp.zeros((8, 8), jnp.float32).at[0, 0].set(lp['vel_b2'][0])
        nw1a = lp['node_W1'][0:HID]
        nw1b = lp['node_W1'][HID:]

        hrow, hcol, rec = _gather_call(H, xc8, row, col, edge_attr, zp16)
        mP, wdP = _edge_call(hrow.reshape(E // 4, 128), hcol.reshape(E // 4, 128),
                             rec.reshape(E // 4, 128),
                             dm, bdw1a, bdw1b, wr, wea, _r8w(lp['edge_b1']),
                             bdw2, _r8w(lp['edge_b2']), bdcw1,
                             _r8w(lp['coord_b1']), bdcw2, sp)
        m = mP.reshape(E, HID)
        wd4 = wdP.reshape(E, HID)
        outm, outw = _scatter_call(m, wd4, row2d, zm, zw)
        outm = jnp.concatenate([outm[:HALF], outm[AROWS:AROWS + HALF]], axis=0)
        outw = jnp.concatenate([outw[:HALF], outw[AROWS:AROWS + HALF]], axis=0)
        H, xc8 = _node_call(H, xc8, vel8, outm, outw,
                            cent8 if li == len(params['layers']) - 1 else zcent8,
                            lp['vel_W1'], _row8(lp['vel_b1']), vw2p, vb2p,
                            nw1a, nw1b, _row8(lp['node_b1']),
                            lp['node_W2'], _row8(lp['node_b2']))

    return xc8[:, :3]


# speedup vs baseline: 2.1522x; 1.0087x over previous
import functools

import jax
import jax.numpy as jnp
import numpy as np
from jax import lax
from jax.experimental import pallas as pl
from jax.experimental.pallas import tpu as pltpu
from jax.experimental.pallas import tpu_sc as plsc

N = 100000
E = 1600000
HID = 32
G = 5
HALF = N // 2
AROWS = 50016
NC, NS, LANES = 2, 16, 16
NW = NC * NS

BLK = 400
SUB = 80
NSUB = BLK // SUB
EPT_G = E // NW
NBLK_G = EPT_G // BLK
SUBB = 128
NBATCH = E // SUBB
QMAX = -(-NBATCH // NS)
ZCH = 521
RPT = AROWS // NS
NZ = RPT // ZCH

BE = 12800
NB = 400

_SC_PARAMS = pltpu.CompilerParams(use_tc_tiling_on_sc=False)


def _silu(v):
    return v * jax.nn.sigmoid(v)


def _gather_body(h_hbm, xc_hbm, row_hbm, col_hbm, ea_hbm, zp_hbm,
                 hrow_hbm, hcol_hbm, rec_hbm,
                 idxr, idxc, hrow_v, hcol_v, xr_v, xcv_v, eaz_v,
                 sem0, sem1, sem2, sem3):
    c = lax.axis_index("c")
    s = lax.axis_index("s")
    wid = s * NC + c
    pltpu.sync_copy(zp_hbm, eaz_v)

    def blk(j, carry):
        base = wid * EPT_G + j * BLK
        pltpu.sync_copy(row_hbm.at[pl.ds(base, BLK)], idxr)
        pltpu.sync_copy(col_hbm.at[pl.ds(base, BLK)], idxc)
        pltpu.sync_copy(ea_hbm.at[pl.ds(base, BLK)], eaz_v.at[pl.ds(0, BLK), pl.ds(0, 4)])
        cps = []
        for t in range(NSUB):
            sl = pl.ds(t * SUB, SUB)
            cps.append(pltpu.async_copy(h_hbm.at[idxr.at[sl]], hrow_v.at[sl], sem0))
            cps.append(pltpu.async_copy(h_hbm.at[idxc.at[sl]], hcol_v.at[sl], sem1))
            cps.append(pltpu.async_copy(xc_hbm.at[idxr.at[sl]], xr_v.at[sl], sem2))
            cps.append(pltpu.async_copy(xc_hbm.at[idxc.at[sl]], xcv_v.at[sl], sem3))
        for cp in cps:
            cp.wait()
        pltpu.sync_copy(hrow_v, hrow_hbm.at[pl.ds(base, BLK)])
        pltpu.sync_copy(hcol_v, hcol_hbm.at[pl.ds(base, BLK)])
        pltpu.sync_copy(xr_v, rec_hbm.at[pl.ds(base, BLK), pl.ds(0, 8)])
        pltpu.sync_copy(xcv_v, rec_hbm.at[pl.ds(base, BLK), pl.ds(8, 8)])
        pltpu.sync_copy(eaz_v, rec_hbm.at[pl.ds(base, BLK), pl.ds(16, 16)])
        return carry

    lax.fori_loop(0, NBLK_G, blk, 0)


def _gather_call(h, xc8, row1d, col1d, ea, zp16):
    mesh = plsc.VectorSubcoreMesh(core_axis_name="c", subcore_axis_name="s",
                                  num_cores=NC, num_subcores=NS)
    f = pl.kernel(
        _gather_body,
        out_type=[jax.ShapeDtypeStruct((E, HID), jnp.float32),
                  jax.ShapeDtypeStruct((E, HID), jnp.float32),
                  jax.ShapeDtypeStruct((E, HID), jnp.float32)],
        mesh=mesh,
        scratch_types=[
            pltpu.VMEM((BLK,), jnp.int32),
            pltpu.VMEM((BLK,), jnp.int32),
            pltpu.VMEM((BLK, HID), jnp.float32),
            pltpu.VMEM((BLK, HID), jnp.float32),
            pltpu.VMEM((BLK, 8), jnp.float32),
            pltpu.VMEM((BLK, 8), jnp.float32),
            pltpu.VMEM((BLK, 16), jnp.float32),
            pltpu.SemaphoreType.DMA,
            pltpu.SemaphoreType.DMA,
            pltpu.SemaphoreType.DMA,
            pltpu.SemaphoreType.DMA,
        ],
        compiler_params=_SC_PARAMS,
    )
    return f(h, xc8, row1d, col1d, ea, zp16)


def _scatter_body(m_hbm, wd_hbm, row2d_hbm, zm_hbm, zw_hbm,
                  outm_hbm, outw_hbm,
                  accm, accw, idx, mv, wv):
    c = lax.axis_index("c")
    s = lax.axis_index("s")

    for q in range(NZ):
        r = s * RPT + q * ZCH
        pltpu.sync_copy(zm_hbm, accm.at[pl.ds(r, ZCH)])
        pltpu.sync_copy(zw_hbm, accw.at[pl.ds(r, ZCH)])
    plsc.subcore_barrier()

    def blk(q, carry):
        b = q * NS + s

        @pl.when(b < NBATCH)
        def _():
            base = b * SUBB
            pltpu.sync_copy(row2d_hbm.at[pl.ds(base, SUBB)], idx)
            for g in range(SUBB // LANES):
                sl = pl.ds(g * LANES, LANES)
                v = idx[sl]
                lv = v - c * HALF
                ok = (lv >= 0) & (lv < HALF)
                idx[sl] = jnp.where(ok, lv, HALF)
            pltpu.sync_copy(m_hbm.at[pl.ds(base, SUBB)], mv)
            pltpu.sync_copy(wd_hbm.at[pl.ds(base, SUBB), pl.ds(0, 8)], wv)
            pltpu.sync_copy(mv, accm.at[idx], add=True)
            pltpu.sync_copy(wv, accw.at[idx], add=True)

        return carry

    lax.fori_loop(0, QMAX, blk, 0)
    plsc.subcore_barrier()

    for q in range(NZ):
        r = s * RPT + q * ZCH
        pltpu.sync_copy(accm.at[pl.ds(r, ZCH)], outm_hbm.at[pl.ds(c * AROWS + r, ZCH)])
        pltpu.sync_copy(accw.at[pl.ds(r, ZCH)], outw_hbm.at[pl.ds(c * AROWS + r, ZCH)])


def _scatter_call(m, wd4, row2d, zm, zw):
    mesh = plsc.VectorSubcoreMesh(core_axis_name="c", subcore_axis_name="s",
                                  num_cores=NC, num_subcores=NS)
    f = pl.kernel(
        _scatter_body,
        out_type=[jax.ShapeDtypeStruct((2 * AROWS, HID), jnp.float32),
                  jax.ShapeDtypeStruct((2 * AROWS, 8), jnp.float32)],
        mesh=mesh,
        scratch_types=[
            pltpu.VMEM_SHARED((AROWS, HID), jnp.float32),
            pltpu.VMEM_SHARED((AROWS, 8), jnp.float32),
            pltpu.VMEM((SUBB,), jnp.int32),
            pltpu.VMEM((SUBB, HID), jnp.float32),
            pltpu.VMEM((SUBB, 8), jnp.float32),
        ],
        compiler_params=_SC_PARAMS,
    )
    return f(m, wd4, row2d, zm, zw)


def _edge_body(hrowP, hcolP, recP,
               dm, bdw1a, bdw1b, wr, wea, b1t, bdw2, b2t, bdcw1, cb1t,
               bdcw2, sp, m_o, wd_o):
    f32 = jnp.float32
    dot = lambda a, b: jnp.dot(a, b, preferred_element_type=f32)
    rec = recP[...]
    d = dot(rec, dm[...])
    dsq = d * d
    z1 = (dot(hrowP[...], bdw1a[...]) + dot(hcolP[...], bdw1b[...])
          + dot(dsq, wr[...]) + dot(rec, wea[...]) + b1t[...][0:1])
    a = _silu(z1)
    m = _silu(dot(a, bdw2[...]) + b2t[...][0:1])
    p = _silu(dot(m, bdcw1[...]) + cb1t[...][0:1])
    cw4 = dot(p, bdcw2[...])
    m_o[...] = m
    cwsp = dot(cw4, sp[...])
    l128 = lax.broadcasted_iota(jnp.int32, (1, 128), 1)
    wd_o[...] = d * cwsp + ((l128 % 32) == 3).astype(f32)


def _edge_call(hrowP, hcolP, recP,
               dm, bdw1a, bdw1b, wr, wea, b1t, bdw2, b2t, bdcw1, cb1t,
               bdcw2, sp):
    nblk = E // BE
    wspec = lambda shp: pl.BlockSpec(shp, lambda i: (0, 0))
    bigspec = pl.BlockSpec((BE // 4, 128), lambda i: (i, 0))
    return pl.pallas_call(
        _edge_body,
        grid=(nblk,),
        in_specs=[
            bigspec, bigspec, bigspec,
            wspec((128, 128)), wspec((128, 128)), wspec((128, 128)),
            wspec((128, 128)), wspec((128, 128)), wspec((8, 128)),
            wspec((128, 128)), wspec((8, 128)), wspec((128, 128)),
            wspec((8, 128)), wspec((128, 128)), wspec((128, 128)),
        ],
        out_specs=[bigspec, bigspec],
        out_shape=[jax.ShapeDtypeStruct((E // 4, 128), jnp.float32),
                   jax.ShapeDtypeStruct((E // 4, 128), jnp.float32)],
    )(hrowP, hcolP, recP,
      dm, bdw1a, bdw1b, wr, wea, b1t, bdw2, b2t, bdcw1, cb1t, bdcw2, sp)


def _node_body(h, xc8, vel8, outm, outw, cent8,
               vw1, vb1, vw2p, vb2p, nw1a, nw1b, nb1, nw2, nb2,
               ho, xco):
    hv = h[...]
    w = outw[...]
    cnt = jnp.maximum(w[:, 3:4], 1.0)
    lane8 = lax.broadcasted_iota(jnp.int32, (1, 8), 1)
    mul = (lane8 < 3).astype(jnp.float32)
    agg8 = w * mul / cnt
    xcv = xc8[...] + agg8
    vz = _silu(jnp.dot(hv, vw1[...], preferred_element_type=jnp.float32) + vb1[...][0:1])
    vw8 = jnp.dot(vz, vw2p[...], preferred_element_type=jnp.float32) + vb2p[...][0:1]
    xcv = xcv + vw8[:, 0:1] * vel8[...]
    xco[...] = xcv + cent8[...]
    magg = outm[...]
    z = _silu(jnp.dot(hv, nw1a[...], preferred_element_type=jnp.float32)
              + jnp.dot(magg, nw1b[...], preferred_element_type=jnp.float32)
              + nb1[...][0:1])
    ho[...] = jnp.dot(z, nw2[...], preferred_element_type=jnp.float32) + nb2[...][0:1]


def _node_call(h, xc8, vel8, outm, outw, cent8,
               vw1, vb1, vw2p, vb2p, nw1a, nw1b, nb1, nw2, nb2):
    nblk = N // NB
    wspec = lambda shp: pl.BlockSpec(shp, lambda i: (0, 0))
    return pl.pallas_call(
        _node_body,
        grid=(nblk,),
        in_specs=[
            pl.BlockSpec((NB, HID), lambda i: (i, 0)),
            pl.BlockSpec((NB, 8), lambda i: (i, 0)),
            pl.BlockSpec((NB, 8), lambda i: (i, 0)),
            pl.BlockSpec((NB, HID), lambda i: (i, 0)),
            pl.BlockSpec((NB, 8), lambda i: (i, 0)),
            pl.BlockSpec((NB, 8), lambda i: (i, 0)),
            wspec((HID, HID)), wspec((8, HID)), wspec((HID, 8)),
            wspec((8, 8)), wspec((HID, HID)), wspec((HID, HID)),
            wspec((8, HID)), wspec((HID, HID)), wspec((8, HID)),
        ],
        out_specs=[
            pl.BlockSpec((NB, HID), lambda i: (i, 0)),
            pl.BlockSpec((NB, 8), lambda i: (i, 0)),
        ],
        out_shape=[jax.ShapeDtypeStruct((N, HID), jnp.float32),
                   jax.ShapeDtypeStruct((N, 8), jnp.float32)],
    )(h, xc8, vel8, outm, outw, cent8,
      vw1, vb1, vw2p, vb2p, nw1a, nw1b, nb1, nw2, nb2)


def _pro_body(h8, x16, embw, embb, ic, cm, h0_o, xcc_o, cent_o):
    h0_o[...] = jnp.dot(h8[...], embw[...], preferred_element_type=jnp.float32) + embb[...][0:1]
    xv = x16[...]
    xcc_o[...] = jnp.dot(xv, ic[...], preferred_element_type=jnp.float32)
    cent_o[...] = jnp.dot(xv, cm[...], preferred_element_type=jnp.float32)


def _pro_call(h8, x16, embw, embb, ic, cm):
    nblk = 100
    wspec = lambda shp: pl.BlockSpec(shp, lambda i: (0, 0))
    return pl.pallas_call(
        _pro_body,
        grid=(nblk,),
        in_specs=[
            pl.BlockSpec((N // nblk, 8), lambda i: (i, 0)),
            pl.BlockSpec((N // G // nblk, 16), lambda i: (i, 0)),
            wspec((8, HID)), wspec((8, HID)), wspec((16, 16)), wspec((16, 16)),
        ],
        out_specs=[
            pl.BlockSpec((N // nblk, HID), lambda i: (i, 0)),
            pl.BlockSpec((N // G // nblk, 16), lambda i: (i, 0)),
            pl.BlockSpec((N // G // nblk, 16), lambda i: (i, 0)),
        ],
        out_shape=[jax.ShapeDtypeStruct((N, HID), jnp.float32),
                   jax.ShapeDtypeStruct((N // G, 16), jnp.float32),
                   jax.ShapeDtypeStruct((N // G, 16), jnp.float32)],
    )(h8, x16, embw, embb, ic, cm)


def _row8(b):
    out = jnp.zeros((8, b.shape[-1]), jnp.float32)
    return out.at[0].set(b)


def kernel(h, x, edges, vel, edge_attr, params):
    row = edges[0]
    col = edges[1]
    row2d = row

    h8 = jnp.pad(h, ((0, 0), (0, 8 - h.shape[1])))
    x16 = jnp.pad(x.reshape(N // G, 3 * G), ((0, 0), (0, 1)))
    vel8 = jnp.pad(vel, ((0, 0), (0, 5)))

    embw = jnp.pad(params['emb_W'], ((0, 8 - params['emb_W'].shape[0]), (0, 0)))
    embb = _row8(params['emb_b'])
    cmat = np.zeros((16, 16), np.float32)
    for i in range(3 * G):
        for j in range(3 * G):
            if i % 3 == j % 3:
                cmat[i, j] = 1.0 / G
    icmat = np.eye(16, dtype=np.float32)
    icmat[15, 15] = 0.0
    icmat = icmat - cmat
    cmat = jnp.asarray(cmat)
    icmat = jnp.asarray(icmat)

    H, xcc16, cent16 = _pro_call(h8, x16, embw, embb, icmat, cmat)
    xc8 = jnp.pad(xcc16[:, :3 * G].reshape(N, 3), ((0, 0), (0, 5)))
    cent8 = jnp.pad(cent16[:, :3 * G].reshape(N, 3), ((0, 0), (0, 5)))
    zcent8 = jnp.zeros_like(cent8)

    zm = jnp.zeros((ZCH, HID), jnp.float32)
    zw = jnp.zeros((ZCH, 8), jnp.float32)

    i4 = jnp.eye(4, dtype=jnp.float32)
    m3c32 = jnp.asarray((np.arange(32) < 3).astype(np.float32).reshape(32, 1))
    dmnp = np.zeros((128, 128), np.float32)
    for g in range(4):
        for k in range(3):
            dmnp[32 * g + k, 32 * g + k] = 1.0
            dmnp[32 * g + k + 8, 32 * g + k] = -1.0
    dm = jnp.asarray(dmnp)
    b3 = jnp.zeros((32, 32), jnp.float32).at[0, 0:3].set(1.0)
    sp = jnp.kron(i4, b3)
    zp16 = jnp.zeros((BLK, 16), jnp.float32)
    _r8w = lambda b: jnp.zeros((8, 128), jnp.float32).at[0].set(jnp.tile(b, 4))

    for li, lp in enumerate(params['layers']):
        bdw1a = jnp.kron(i4, lp['edge_W1'][0:HID])
        bdw1b = jnp.kron(i4, lp['edge_W1'][HID:2 * HID])
        wr = jnp.kron(i4, m3c32 @ lp['edge_W1'][2 * HID:2 * HID + 1])
        wea = jnp.kron(i4, jnp.zeros((HID, HID), jnp.float32
                                     ).at[16:20].set(lp['edge_W1'][2 * HID + 1:]))
        bdw2 = jnp.kron(i4, lp['edge_W2'])
        bdcw1 = jnp.kron(i4, lp['coord_W1'])
        bdcw2 = jnp.kron(i4, jnp.pad(lp['coord_W2'], ((0, 0), (0, 31))))
        vw2p = jnp.pad(lp['vel_W2'], ((0, 0), (0, 7)))
        vb2p = jnp.zeros((8, 8), jnp.float32).at[0, 0].set(lp['vel_b2'][0])
        nw1a = lp['node_W1'][0:HID]
        nw1b = lp['node_W1'][HID:]

        hrow, hcol, rec = _gather_call(H, xc8, row, col, edge_attr, zp16)
        mP, wdP = _edge_call(hrow.reshape(E // 4, 128), hcol.reshape(E // 4, 128),
                             rec.reshape(E // 4, 128),
                             dm, bdw1a, bdw1b, wr, wea, _r8w(lp['edge_b1']),
                             bdw2, _r8w(lp['edge_b2']), bdcw1,
                             _r8w(lp['coord_b1']), bdcw2, sp)
        m = mP.reshape(E, HID)
        wd4 = wdP.reshape(E, HID)
        outm, outw = _scatter_call(m, wd4, row2d, zm, zw)
        outm = jnp.concatenate([outm[:HALF], outm[AROWS:AROWS + HALF]], axis=0)
        outw = jnp.concatenate([outw[:HALF], outw[AROWS:AROWS + HALF]], axis=0)
        H, xc8 = _node_call(H, xc8, vel8, outm, outw,
                            cent8 if li == len(params['layers']) - 1 else zcent8,
                            lp['vel_W1'], _row8(lp['vel_b1']), vw2p, vb2p,
                            nw1a, nw1b, _row8(lp['node_b1']),
                            lp['node_W2'], _row8(lp['node_b2']))

    return xc8[:, :3]


# speedup vs baseline: 2.2762x; 1.0576x over previous
import functools

import jax
import jax.numpy as jnp
import numpy as np
from jax import lax
from jax.experimental import pallas as pl
from jax.experimental.pallas import tpu as pltpu
from jax.experimental.pallas import tpu_sc as plsc

N = 100000
E = 1600000
HID = 32
G = 5
HALF = N // 2
AROWS = 50016
NC, NS, LANES = 2, 16, 16
NW = NC * NS

BLK = 400
SUB = 80
NSUB = BLK // SUB
EPT_G = E // NW
NBLK_G = EPT_G // BLK
SUBB = 128
NBATCH = E // SUBB
QMAX = -(-NBATCH // NS)
ZCH = 521
RPT = AROWS // NS
NZ = RPT // ZCH

BE = 12800
NB = 400

_SC_PARAMS = pltpu.CompilerParams(use_tc_tiling_on_sc=False)


def _silu(v):
    return v * jax.nn.sigmoid(v)


def _fire_gathers(h_hbm, xc_hbm, idxr, idxc, hr, hc, xr, xcv, s0, s1, s2, s3):
    for t in range(NSUB):
        sl = pl.ds(t * SUB, SUB)
        pltpu.async_copy(h_hbm.at[idxr.at[sl]], hr.at[sl], s0)
        pltpu.async_copy(h_hbm.at[idxc.at[sl]], hc.at[sl], s1)
        pltpu.async_copy(xc_hbm.at[idxr.at[sl]], xr.at[sl], s2)
        pltpu.async_copy(xc_hbm.at[idxc.at[sl]], xcv.at[sl], s3)


def _wait_gathers(h_hbm, xc_hbm, idxr, idxc, hr, hc, xr, xcv, s0, s1, s2, s3):
    for t in range(NSUB):
        sl = pl.ds(t * SUB, SUB)
        pltpu.make_async_copy(h_hbm.at[idxr.at[sl]], hr.at[sl], s0).wait()
        pltpu.make_async_copy(h_hbm.at[idxc.at[sl]], hc.at[sl], s1).wait()
        pltpu.make_async_copy(xc_hbm.at[idxr.at[sl]], xr.at[sl], s2).wait()
        pltpu.make_async_copy(xc_hbm.at[idxc.at[sl]], xcv.at[sl], s3).wait()


def _write_dsts(base, hrow_hbm, hcol_hbm, rec_hbm):
    return [
        hrow_hbm.at[pl.ds(base, BLK)],
        hcol_hbm.at[pl.ds(base, BLK)],
        rec_hbm.at[pl.ds(base, BLK), pl.ds(0, 8)],
        rec_hbm.at[pl.ds(base, BLK), pl.ds(8, 8)],
        rec_hbm.at[pl.ds(base, BLK), pl.ds(16, 16)],
    ]


def _gather_body(h_hbm, xc_hbm, row_hbm, col_hbm, ea_hbm, zp_hbm,
                 hrow_hbm, hcol_hbm, rec_hbm,
                 idxrA, idxcA, hrA, hcA, xrA, xcvA, eazA,
                 idxrB, idxcB, hrB, hcB, xrB, xcvB, eazB,
                 sA0, sA1, sA2, sA3, sB0, sB1, sB2, sB3, wsA, wsB):
    c = lax.axis_index("c")
    s = lax.axis_index("s")
    wid = s * NC + c
    pltpu.sync_copy(zp_hbm, eazA)
    pltpu.sync_copy(zp_hbm, eazB)
    bufsA = (idxrA, idxcA, hrA, hcA, xrA, xcvA, eazA)
    bufsB = (idxrB, idxcB, hrB, hcB, xrB, xcvB, eazB)
    semsA = (sA0, sA1, sA2, sA3)
    semsB = (sB0, sB1, sB2, sB3)

    def load_and_fire(base, bufs, sems):
        idxr, idxc, hr, hc, xr, xcv, eaz = bufs
        pltpu.sync_copy(row_hbm.at[pl.ds(base, BLK)], idxr)
        pltpu.sync_copy(col_hbm.at[pl.ds(base, BLK)], idxc)
        pltpu.sync_copy(ea_hbm.at[pl.ds(base, BLK)],
                        eaz.at[pl.ds(0, BLK), pl.ds(0, 4)])
        _fire_gathers(h_hbm, xc_hbm, idxr, idxc, hr, hc, xr, xcv, *sems)

    def finish(base, bufs, sems, wsem):
        idxr, idxc, hr, hc, xr, xcv, eaz = bufs
        _wait_gathers(h_hbm, xc_hbm, idxr, idxc, hr, hc, xr, xcv, *sems)
        srcs = [hr, hc, xr, xcv, eaz]
        for sref, dref in zip(srcs, _write_dsts(base, hrow_hbm, hcol_hbm, rec_hbm)):
            pltpu.async_copy(sref, dref, wsem)

    def wait_writes(base, bufs, wsem):
        idxr, idxc, hr, hc, xr, xcv, eaz = bufs
        srcs = [hr, hc, xr, xcv, eaz]
        for sref, dref in zip(srcs, _write_dsts(base, hrow_hbm, hcol_hbm, rec_hbm)):
            pltpu.make_async_copy(sref, dref, wsem).wait()

    npair = NBLK_G // 2

    def pair(k, carry):
        baseA = wid * EPT_G + (2 * k) * BLK
        baseB = wid * EPT_G + (2 * k + 1) * BLK

        @pl.when(k > 0)
        def _():
            wait_writes(baseA, bufsA, wsA)

        load_and_fire(baseA, bufsA, semsA)

        @pl.when(k > 0)
        def _():
            wait_writes(baseB, bufsB, wsB)

        load_and_fire(baseB, bufsB, semsB)
        finish(baseA, bufsA, semsA, wsA)
        finish(baseB, bufsB, semsB, wsB)
        return carry

    lax.fori_loop(0, npair, pair, 0)
    lastA = wid * EPT_G + (2 * npair - 2) * BLK
    lastB = wid * EPT_G + (2 * npair - 1) * BLK
    wait_writes(lastA, bufsA, wsA)
    wait_writes(lastB, bufsB, wsB)
    baseT = wid * EPT_G + (2 * npair) * BLK
    load_and_fire(baseT, bufsA, semsA)
    finish(baseT, bufsA, semsA, wsA)
    wait_writes(baseT, bufsA, wsA)


def _gather_call(h, xc8, row1d, col1d, ea, zp16):
    mesh = plsc.VectorSubcoreMesh(core_axis_name="c", subcore_axis_name="s",
                                  num_cores=NC, num_subcores=NS)
    dbl = lambda: [
        pltpu.VMEM((BLK,), jnp.int32),
        pltpu.VMEM((BLK,), jnp.int32),
        pltpu.VMEM((BLK, HID), jnp.float32),
        pltpu.VMEM((BLK, HID), jnp.float32),
        pltpu.VMEM((BLK, 8), jnp.float32),
        pltpu.VMEM((BLK, 8), jnp.float32),
        pltpu.VMEM((BLK, 16), jnp.float32),
    ]
    f = pl.kernel(
        _gather_body,
        out_type=[jax.ShapeDtypeStruct((E, HID), jnp.float32),
                  jax.ShapeDtypeStruct((E, HID), jnp.float32),
                  jax.ShapeDtypeStruct((E, HID), jnp.float32)],
        mesh=mesh,
        scratch_types=dbl() + dbl() + [pltpu.SemaphoreType.DMA] * 10,
        compiler_params=_SC_PARAMS,
    )
    return f(h, xc8, row1d, col1d, ea, zp16)


def _scatter_body(m_hbm, wd_hbm, row2d_hbm, zm_hbm, zw_hbm,
                  outm_hbm, outw_hbm,
                  accm, accw, idx, mv, wv):
    c = lax.axis_index("c")
    s = lax.axis_index("s")

    for q in range(NZ):
        r = s * RPT + q * ZCH
        pltpu.sync_copy(zm_hbm, accm.at[pl.ds(r, ZCH)])
        pltpu.sync_copy(zw_hbm, accw.at[pl.ds(r, ZCH)])
    plsc.subcore_barrier()

    def blk(q, carry):
        b = q * NS + s

        @pl.when(b < NBATCH)
        def _():
            base = b * SUBB
            pltpu.sync_copy(row2d_hbm.at[pl.ds(base, SUBB)], idx)
            for g in range(SUBB // LANES):
                sl = pl.ds(g * LANES, LANES)
                v = idx[sl]
                lv = v - c * HALF
                ok = (lv >= 0) & (lv < HALF)
                idx[sl] = jnp.where(ok, lv, HALF)
            pltpu.sync_copy(m_hbm.at[pl.ds(base, SUBB)], mv)
            pltpu.sync_copy(wd_hbm.at[pl.ds(base, SUBB), pl.ds(0, 8)], wv)
            pltpu.sync_copy(mv, accm.at[idx], add=True)
            pltpu.sync_copy(wv, accw.at[idx], add=True)

        return carry

    lax.fori_loop(0, QMAX, blk, 0)
    plsc.subcore_barrier()

    for q in range(NZ):
        r = s * RPT + q * ZCH
        pltpu.sync_copy(accm.at[pl.ds(r, ZCH)], outm_hbm.at[pl.ds(c * AROWS + r, ZCH)])
        pltpu.sync_copy(accw.at[pl.ds(r, ZCH)], outw_hbm.at[pl.ds(c * AROWS + r, ZCH)])


def _scatter_call(m, wd4, row2d, zm, zw):
    mesh = plsc.VectorSubcoreMesh(core_axis_name="c", subcore_axis_name="s",
                                  num_cores=NC, num_subcores=NS)
    f = pl.kernel(
        _scatter_body,
        out_type=[jax.ShapeDtypeStruct((2 * AROWS, HID), jnp.float32),
                  jax.ShapeDtypeStruct((2 * AROWS, 8), jnp.float32)],
        mesh=mesh,
        scratch_types=[
            pltpu.VMEM_SHARED((AROWS, HID), jnp.float32),
            pltpu.VMEM_SHARED((AROWS, 8), jnp.float32),
            pltpu.VMEM((SUBB,), jnp.int32),
            pltpu.VMEM((SUBB, HID), jnp.float32),
            pltpu.VMEM((SUBB, 8), jnp.float32),
        ],
        compiler_params=_SC_PARAMS,
    )
    return f(m, wd4, row2d, zm, zw)


def _edge_body(hrowP, hcolP, recP,
               dm, bdw1a, bdw1b, wr, wea, b1t, bdw2, b2t, bdcw1, cb1t,
               bdcw2, sp, m_o, wd_o):
    f32 = jnp.float32
    dot = lambda a, b: jnp.dot(a, b, preferred_element_type=f32)
    rec = recP[...]
    d = dot(rec, dm[...])
    dsq = d * d
    z1 = (dot(hrowP[...], bdw1a[...]) + dot(hcolP[...], bdw1b[...])
          + dot(dsq, wr[...]) + dot(rec, wea[...]) + b1t[...][0:1])
    a = _silu(z1)
    m = _silu(dot(a, bdw2[...]) + b2t[...][0:1])
    p = _silu(dot(m, bdcw1[...]) + cb1t[...][0:1])
    cw4 = dot(p, bdcw2[...])
    m_o[...] = m
    cwsp = dot(cw4, sp[...])
    l128 = lax.broadcasted_iota(jnp.int32, (1, 128), 1)
    wd_o[...] = d * cwsp + ((l128 % 32) == 3).astype(f32)


def _edge_call(hrowP, hcolP, recP,
               dm, bdw1a, bdw1b, wr, wea, b1t, bdw2, b2t, bdcw1, cb1t,
               bdcw2, sp):
    nblk = E // BE
    wspec = lambda shp: pl.BlockSpec(shp, lambda i: (0, 0))
    bigspec = pl.BlockSpec((BE // 4, 128), lambda i: (i, 0))
    return pl.pallas_call(
        _edge_body,
        grid=(nblk,),
        in_specs=[
            bigspec, bigspec, bigspec,
            wspec((128, 128)), wspec((128, 128)), wspec((128, 128)),
            wspec((128, 128)), wspec((128, 128)), wspec((8, 128)),
            wspec((128, 128)), wspec((8, 128)), wspec((128, 128)),
            wspec((8, 128)), wspec((128, 128)), wspec((128, 128)),
        ],
        out_specs=[bigspec, bigspec],
        out_shape=[jax.ShapeDtypeStruct((E // 4, 128), jnp.float32),
                   jax.ShapeDtypeStruct((E // 4, 128), jnp.float32)],
    )(hrowP, hcolP, recP,
      dm, bdw1a, bdw1b, wr, wea, b1t, bdw2, b2t, bdcw1, cb1t, bdcw2, sp)


def _node_body(h, xc8, vel8, outm, outw, cent8,
               vw1, vb1, vw2p, vb2p, nw1a, nw1b, nb1, nw2, nb2,
               ho, xco):
    hv = h[...]
    w = outw[...]
    cnt = jnp.maximum(w[:, 3:4], 1.0)
    lane8 = lax.broadcasted_iota(jnp.int32, (1, 8), 1)
    mul = (lane8 < 3).astype(jnp.float32)
    agg8 = w * mul / cnt
    xcv = xc8[...] + agg8
    vz = _silu(jnp.dot(hv, vw1[...], preferred_element_type=jnp.float32) + vb1[...][0:1])
    vw8 = jnp.dot(vz, vw2p[...], preferred_element_type=jnp.float32) + vb2p[...][0:1]
    xcv = xcv + vw8[:, 0:1] * vel8[...]
    xco[...] = xcv + cent8[...]
    magg = outm[...]
    z = _silu(jnp.dot(hv, nw1a[...], preferred_element_type=jnp.float32)
              + jnp.dot(magg, nw1b[...], preferred_element_type=jnp.float32)
              + nb1[...][0:1])
    ho[...] = jnp.dot(z, nw2[...], preferred_element_type=jnp.float32) + nb2[...][0:1]


def _node_call(h, xc8, vel8, outm, outw, cent8,
               vw1, vb1, vw2p, vb2p, nw1a, nw1b, nb1, nw2, nb2):
    nblk = N // NB
    wspec = lambda shp: pl.BlockSpec(shp, lambda i: (0, 0))
    return pl.pallas_call(
        _node_body,
        grid=(nblk,),
        in_specs=[
            pl.BlockSpec((NB, HID), lambda i: (i, 0)),
            pl.BlockSpec((NB, 8), lambda i: (i, 0)),
            pl.BlockSpec((NB, 8), lambda i: (i, 0)),
            pl.BlockSpec((NB, HID), lambda i: (i, 0)),
            pl.BlockSpec((NB, 8), lambda i: (i, 0)),
            pl.BlockSpec((NB, 8), lambda i: (i, 0)),
            wspec((HID, HID)), wspec((8, HID)), wspec((HID, 8)),
            wspec((8, 8)), wspec((HID, HID)), wspec((HID, HID)),
            wspec((8, HID)), wspec((HID, HID)), wspec((8, HID)),
        ],
        out_specs=[
            pl.BlockSpec((NB, HID), lambda i: (i, 0)),
            pl.BlockSpec((NB, 8), lambda i: (i, 0)),
        ],
        out_shape=[jax.ShapeDtypeStruct((N, HID), jnp.float32),
                   jax.ShapeDtypeStruct((N, 8), jnp.float32)],
    )(h, xc8, vel8, outm, outw, cent8,
      vw1, vb1, vw2p, vb2p, nw1a, nw1b, nb1, nw2, nb2)


def _pro_body(h8, x16, embw, embb, ic, cm, h0_o, xcc_o, cent_o):
    h0_o[...] = jnp.dot(h8[...], embw[...], preferred_element_type=jnp.float32) + embb[...][0:1]
    xv = x16[...]
    xcc_o[...] = jnp.dot(xv, ic[...], preferred_element_type=jnp.float32)
    cent_o[...] = jnp.dot(xv, cm[...], preferred_element_type=jnp.float32)


def _pro_call(h8, x16, embw, embb, ic, cm):
    nblk = 100
    wspec = lambda shp: pl.BlockSpec(shp, lambda i: (0, 0))
    return pl.pallas_call(
        _pro_body,
        grid=(nblk,),
        in_specs=[
            pl.BlockSpec((N // nblk, 8), lambda i: (i, 0)),
            pl.BlockSpec((N // G // nblk, 16), lambda i: (i, 0)),
            wspec((8, HID)), wspec((8, HID)), wspec((16, 16)), wspec((16, 16)),
        ],
        out_specs=[
            pl.BlockSpec((N // nblk, HID), lambda i: (i, 0)),
            pl.BlockSpec((N // G // nblk, 16), lambda i: (i, 0)),
            pl.BlockSpec((N // G // nblk, 16), lambda i: (i, 0)),
        ],
        out_shape=[jax.ShapeDtypeStruct((N, HID), jnp.float32),
                   jax.ShapeDtypeStruct((N // G, 16), jnp.float32),
                   jax.ShapeDtypeStruct((N // G, 16), jnp.float32)],
    )(h8, x16, embw, embb, ic, cm)


def _row8(b):
    out = jnp.zeros((8, b.shape[-1]), jnp.float32)
    return out.at[0].set(b)


def kernel(h, x, edges, vel, edge_attr, params):
    row = edges[0]
    col = edges[1]
    row2d = row

    h8 = jnp.pad(h, ((0, 0), (0, 8 - h.shape[1])))
    x16 = jnp.pad(x.reshape(N // G, 3 * G), ((0, 0), (0, 1)))
    vel8 = jnp.pad(vel, ((0, 0), (0, 5)))

    embw = jnp.pad(params['emb_W'], ((0, 8 - params['emb_W'].shape[0]), (0, 0)))
    embb = _row8(params['emb_b'])
    cmat = np.zeros((16, 16), np.float32)
    for i in range(3 * G):
        for j in range(3 * G):
            if i % 3 == j % 3:
                cmat[i, j] = 1.0 / G
    icmat = np.eye(16, dtype=np.float32)
    icmat[15, 15] = 0.0
    icmat = icmat - cmat
    cmat = jnp.asarray(cmat)
    icmat = jnp.asarray(icmat)

    H, xcc16, cent16 = _pro_call(h8, x16, embw, embb, icmat, cmat)
    xc8 = jnp.pad(xcc16[:, :3 * G].reshape(N, 3), ((0, 0), (0, 5)))
    cent8 = jnp.pad(cent16[:, :3 * G].reshape(N, 3), ((0, 0), (0, 5)))
    zcent8 = jnp.zeros_like(cent8)

    zm = jnp.zeros((ZCH, HID), jnp.float32)
    zw = jnp.zeros((ZCH, 8), jnp.float32)

    i4 = jnp.eye(4, dtype=jnp.float32)
    m3c32 = jnp.asarray((np.arange(32) < 3).astype(np.float32).reshape(32, 1))
    dmnp = np.zeros((128, 128), np.float32)
    for g in range(4):
        for k in range(3):
            dmnp[32 * g + k, 32 * g + k] = 1.0
            dmnp[32 * g + k + 8, 32 * g + k] = -1.0
    dm = jnp.asarray(dmnp)
    b3 = jnp.zeros((32, 32), jnp.float32).at[0, 0:3].set(1.0)
    sp = jnp.kron(i4, b3)
    zp16 = jnp.zeros((BLK, 16), jnp.float32)
    _r8w = lambda b: jnp.zeros((8, 128), jnp.float32).at[0].set(jnp.tile(b, 4))

    for li, lp in enumerate(params['layers']):
        bdw1a = jnp.kron(i4, lp['edge_W1'][0:HID])
        bdw1b = jnp.kron(i4, lp['edge_W1'][HID:2 * HID])
        wr = jnp.kron(i4, m3c32 @ lp['edge_W1'][2 * HID:2 * HID + 1])
        wea = jnp.kron(i4, jnp.zeros((HID, HID), jnp.float32
                                     ).at[16:20].set(lp['edge_W1'][2 * HID + 1:]))
        bdw2 = jnp.kron(i4, lp['edge_W2'])
        bdcw1 = jnp.kron(i4, lp['coord_W1'])
        bdcw2 = jnp.kron(i4, jnp.pad(lp['coord_W2'], ((0, 0), (0, 31))))
        vw2p = jnp.pad(lp['vel_W2'], ((0, 0), (0, 7)))
        vb2p = jnp.zeros((8, 8), jnp.float32).at[0, 0].set(lp['vel_b2'][0])
        nw1a = lp['node_W1'][0:HID]
        nw1b = lp['node_W1'][HID:]

        hrow, hcol, rec = _gather_call(H, xc8, row, col, edge_attr, zp16)
        mP, wdP = _edge_call(hrow.reshape(E // 4, 128), hcol.reshape(E // 4, 128),
                             rec.reshape(E // 4, 128),
                             dm, bdw1a, bdw1b, wr, wea, _r8w(lp['edge_b1']),
                             bdw2, _r8w(lp['edge_b2']), bdcw1,
                             _r8w(lp['coord_b1']), bdcw2, sp)
        m = mP.reshape(E, HID)
        wd4 = wdP.reshape(E, HID)
        outm, outw = _scatter_call(m, wd4, row2d, zm, zw)
        outm = jnp.concatenate([outm[:HALF], outm[AROWS:AROWS + HALF]], axis=0)
        outw = jnp.concatenate([outw[:HALF], outw[AROWS:AROWS + HALF]], axis=0)
        H, xc8 = _node_call(H, xc8, vel8, outm, outw,
                            cent8 if li == len(params['layers']) - 1 else zcent8,
                            lp['vel_W1'], _row8(lp['vel_b1']), vw2p, vb2p,
                            nw1a, nw1b, _row8(lp['node_b1']),
                            lp['node_W2'], _row8(lp['node_b2']))

    return xc8[:, :3]


# speedup vs baseline: 2.3427x; 1.0292x over previous
import functools

import jax
import jax.numpy as jnp
import numpy as np
from jax import lax
from jax.experimental import pallas as pl
from jax.experimental.pallas import tpu as pltpu
from jax.experimental.pallas import tpu_sc as plsc

N = 100000
E = 1600000
HID = 32
G = 5
HALF = N // 2
AROWS = 50016
NC, NS, LANES = 2, 16, 16
NW = NC * NS

BLK = 400
SUB = 80
NSUB = BLK // SUB
EPT_G = E // NW
NBLK_G = EPT_G // BLK
SUBB = 128
NBATCH = E // SUBB
QMAX = -(-NBATCH // NS)
ZCH = 521
RPT = AROWS // NS
NZ = RPT // ZCH

BE = 12800
NB = 2000

_SC_PARAMS = pltpu.CompilerParams(use_tc_tiling_on_sc=False)


def _silu(v):
    return v * jax.nn.sigmoid(v)


def _fire_gathers(h_hbm, xc_hbm, idxr, idxc, hr, hc, xr, xcv, s0, s1, s2, s3):
    for t in range(NSUB):
        sl = pl.ds(t * SUB, SUB)
        pltpu.async_copy(h_hbm.at[idxr.at[sl]], hr.at[sl], s0)
        pltpu.async_copy(h_hbm.at[idxc.at[sl]], hc.at[sl], s1)
        pltpu.async_copy(xc_hbm.at[idxr.at[sl]], xr.at[sl], s2)
        pltpu.async_copy(xc_hbm.at[idxc.at[sl]], xcv.at[sl], s3)


def _wait_gathers(h_hbm, xc_hbm, idxr, idxc, hr, hc, xr, xcv, s0, s1, s2, s3):
    for t in range(NSUB):
        sl = pl.ds(t * SUB, SUB)
        pltpu.make_async_copy(h_hbm.at[idxr.at[sl]], hr.at[sl], s0).wait()
        pltpu.make_async_copy(h_hbm.at[idxc.at[sl]], hc.at[sl], s1).wait()
        pltpu.make_async_copy(xc_hbm.at[idxr.at[sl]], xr.at[sl], s2).wait()
        pltpu.make_async_copy(xc_hbm.at[idxc.at[sl]], xcv.at[sl], s3).wait()


def _write_dsts(base, hrow_hbm, hcol_hbm, rec_hbm):
    return [
        hrow_hbm.at[pl.ds(base, BLK)],
        hcol_hbm.at[pl.ds(base, BLK)],
        rec_hbm.at[pl.ds(base, BLK), pl.ds(0, 8)],
        rec_hbm.at[pl.ds(base, BLK), pl.ds(8, 8)],
        rec_hbm.at[pl.ds(base, BLK), pl.ds(16, 16)],
    ]


def _gather_body(h_hbm, xc_hbm, row_hbm, col_hbm, ea_hbm, zp_hbm,
                 hrow_hbm, hcol_hbm, rec_hbm,
                 idxrA, idxcA, hrA, hcA, xrA, xcvA, eazA,
                 idxrB, idxcB, hrB, hcB, xrB, xcvB, eazB,
                 sA0, sA1, sA2, sA3, sB0, sB1, sB2, sB3, wsA, wsB):
    c = lax.axis_index("c")
    s = lax.axis_index("s")
    wid = s * NC + c
    pltpu.sync_copy(zp_hbm, eazA)
    pltpu.sync_copy(zp_hbm, eazB)
    bufsA = (idxrA, idxcA, hrA, hcA, xrA, xcvA, eazA)
    bufsB = (idxrB, idxcB, hrB, hcB, xrB, xcvB, eazB)
    semsA = (sA0, sA1, sA2, sA3)
    semsB = (sB0, sB1, sB2, sB3)

    def load_and_fire(base, bufs, sems):
        idxr, idxc, hr, hc, xr, xcv, eaz = bufs
        pltpu.sync_copy(row_hbm.at[pl.ds(base, BLK)], idxr)
        pltpu.sync_copy(col_hbm.at[pl.ds(base, BLK)], idxc)
        pltpu.sync_copy(ea_hbm.at[pl.ds(base, BLK)],
                        eaz.at[pl.ds(0, BLK), pl.ds(0, 4)])
        _fire_gathers(h_hbm, xc_hbm, idxr, idxc, hr, hc, xr, xcv, *sems)

    def finish(base, bufs, sems, wsem):
        idxr, idxc, hr, hc, xr, xcv, eaz = bufs
        _wait_gathers(h_hbm, xc_hbm, idxr, idxc, hr, hc, xr, xcv, *sems)
        srcs = [hr, hc, xr, xcv, eaz]
        for sref, dref in zip(srcs, _write_dsts(base, hrow_hbm, hcol_hbm, rec_hbm)):
            pltpu.async_copy(sref, dref, wsem)

    def wait_writes(base, bufs, wsem):
        idxr, idxc, hr, hc, xr, xcv, eaz = bufs
        srcs = [hr, hc, xr, xcv, eaz]
        for sref, dref in zip(srcs, _write_dsts(base, hrow_hbm, hcol_hbm, rec_hbm)):
            pltpu.make_async_copy(sref, dref, wsem).wait()

    npair = NBLK_G // 2

    def pair(k, carry):
        baseA = wid * EPT_G + (2 * k) * BLK
        baseB = wid * EPT_G + (2 * k + 1) * BLK

        @pl.when(k > 0)
        def _():
            wait_writes(baseA, bufsA, wsA)

        load_and_fire(baseA, bufsA, semsA)

        @pl.when(k > 0)
        def _():
            wait_writes(baseB, bufsB, wsB)

        load_and_fire(baseB, bufsB, semsB)
        finish(baseA, bufsA, semsA, wsA)
        finish(baseB, bufsB, semsB, wsB)
        return carry

    lax.fori_loop(0, npair, pair, 0)
    lastA = wid * EPT_G + (2 * npair - 2) * BLK
    lastB = wid * EPT_G + (2 * npair - 1) * BLK
    wait_writes(lastA, bufsA, wsA)
    wait_writes(lastB, bufsB, wsB)
    baseT = wid * EPT_G + (2 * npair) * BLK
    load_and_fire(baseT, bufsA, semsA)
    finish(baseT, bufsA, semsA, wsA)
    wait_writes(baseT, bufsA, wsA)


def _gather_call(h, xc8, row1d, col1d, ea, zp16):
    mesh = plsc.VectorSubcoreMesh(core_axis_name="c", subcore_axis_name="s",
                                  num_cores=NC, num_subcores=NS)
    dbl = lambda: [
        pltpu.VMEM((BLK,), jnp.int32),
        pltpu.VMEM((BLK,), jnp.int32),
        pltpu.VMEM((BLK, HID), jnp.float32),
        pltpu.VMEM((BLK, HID), jnp.float32),
        pltpu.VMEM((BLK, 8), jnp.float32),
        pltpu.VMEM((BLK, 8), jnp.float32),
        pltpu.VMEM((BLK, 16), jnp.float32),
    ]
    f = pl.kernel(
        _gather_body,
        out_type=[jax.ShapeDtypeStruct((E, HID), jnp.float32),
                  jax.ShapeDtypeStruct((E, HID), jnp.float32),
                  jax.ShapeDtypeStruct((E, HID), jnp.float32)],
        mesh=mesh,
        scratch_types=dbl() + dbl() + [pltpu.SemaphoreType.DMA] * 10,
        compiler_params=_SC_PARAMS,
    )
    return f(h, xc8, row1d, col1d, ea, zp16)


def _scatter_body(m_hbm, wd_hbm, row2d_hbm, zm_hbm, zw_hbm,
                  outm_hbm, outw_hbm,
                  accm, accw, idx, mv, wv):
    c = lax.axis_index("c")
    s = lax.axis_index("s")

    for q in range(NZ):
        r = s * RPT + q * ZCH
        pltpu.sync_copy(zm_hbm, accm.at[pl.ds(r, ZCH)])
        pltpu.sync_copy(zw_hbm, accw.at[pl.ds(r, ZCH)])
    plsc.subcore_barrier()

    def blk(q, carry):
        b = q * NS + s

        @pl.when(b < NBATCH)
        def _():
            base = b * SUBB
            pltpu.sync_copy(row2d_hbm.at[pl.ds(base, SUBB)], idx)
            for g in range(SUBB // LANES):
                sl = pl.ds(g * LANES, LANES)
                v = idx[sl]
                lv = v - c * HALF
                ok = (lv >= 0) & (lv < HALF)
                idx[sl] = jnp.where(ok, lv, HALF)
            pltpu.sync_copy(m_hbm.at[pl.ds(base, SUBB)], mv)
            pltpu.sync_copy(wd_hbm.at[pl.ds(base, SUBB), pl.ds(0, 8)], wv)
            pltpu.sync_copy(mv, accm.at[idx], add=True)
            pltpu.sync_copy(wv, accw.at[idx], add=True)

        return carry

    lax.fori_loop(0, QMAX, blk, 0)
    plsc.subcore_barrier()

    for q in range(NZ):
        r = s * RPT + q * ZCH
        pltpu.sync_copy(accm.at[pl.ds(r, ZCH)], outm_hbm.at[pl.ds(c * AROWS + r, ZCH)])
        pltpu.sync_copy(accw.at[pl.ds(r, ZCH)], outw_hbm.at[pl.ds(c * AROWS + r, ZCH)])


def _scatter_call(m, wd4, row2d, zm, zw):
    mesh = plsc.VectorSubcoreMesh(core_axis_name="c", subcore_axis_name="s",
                                  num_cores=NC, num_subcores=NS)
    f = pl.kernel(
        _scatter_body,
        out_type=[jax.ShapeDtypeStruct((2 * AROWS, HID), jnp.float32),
                  jax.ShapeDtypeStruct((2 * AROWS, 8), jnp.float32)],
        mesh=mesh,
        scratch_types=[
            pltpu.VMEM_SHARED((AROWS, HID), jnp.float32),
            pltpu.VMEM_SHARED((AROWS, 8), jnp.float32),
            pltpu.VMEM((SUBB,), jnp.int32),
            pltpu.VMEM((SUBB, HID), jnp.float32),
            pltpu.VMEM((SUBB, 8), jnp.float32),
        ],
        compiler_params=_SC_PARAMS,
    )
    return f(m, wd4, row2d, zm, zw)


def _edge_body(hrowP, hcolP, recP,
               dm, bdw1a, bdw1b, wr, wea, b1t, bdw2, b2t, bdcw1, cb1t,
               bdcw2, sp, m_o, wd_o):
    f32 = jnp.float32
    dot = lambda a, b: jnp.dot(a, b, preferred_element_type=f32)
    rec = recP[...]
    d = dot(rec, dm[...])
    dsq = d * d
    z1 = (dot(hrowP[...], bdw1a[...]) + dot(hcolP[...], bdw1b[...])
          + dot(dsq, wr[...]) + dot(rec, wea[...]) + b1t[...][0:1])
    a = _silu(z1)
    m = _silu(dot(a, bdw2[...]) + b2t[...][0:1])
    p = _silu(dot(m, bdcw1[...]) + cb1t[...][0:1])
    cw4 = dot(p, bdcw2[...])
    m_o[...] = m
    cwsp = dot(cw4, sp[...])
    l128 = lax.broadcasted_iota(jnp.int32, (1, 128), 1)
    wd_o[...] = d * cwsp + ((l128 % 32) == 3).astype(f32)


def _edge_call(hrowP, hcolP, recP,
               dm, bdw1a, bdw1b, wr, wea, b1t, bdw2, b2t, bdcw1, cb1t,
               bdcw2, sp):
    nblk = E // BE
    wspec = lambda shp: pl.BlockSpec(shp, lambda i: (0, 0))
    bigspec = pl.BlockSpec((BE // 4, 128), lambda i: (i, 0))
    return pl.pallas_call(
        _edge_body,
        grid=(nblk,),
        in_specs=[
            bigspec, bigspec, bigspec,
            wspec((128, 128)), wspec((128, 128)), wspec((128, 128)),
            wspec((128, 128)), wspec((128, 128)), wspec((8, 128)),
            wspec((128, 128)), wspec((8, 128)), wspec((128, 128)),
            wspec((8, 128)), wspec((128, 128)), wspec((128, 128)),
        ],
        out_specs=[bigspec, bigspec],
        out_shape=[jax.ShapeDtypeStruct((E // 4, 128), jnp.float32),
                   jax.ShapeDtypeStruct((E // 4, 128), jnp.float32)],
    )(hrowP, hcolP, recP,
      dm, bdw1a, bdw1b, wr, wea, b1t, bdw2, b2t, bdcw1, cb1t, bdcw2, sp)


def _node_body(h, xc8, vel8, outm, outw, cent8,
               vw1, vb1, vw2p, vb2p, nw1a, nw1b, nb1, nw2, nb2,
               ho, xco):
    hv = h[...]
    w = outw[...]
    cnt = jnp.maximum(w[:, 3:4], 1.0)
    lane8 = lax.broadcasted_iota(jnp.int32, (1, 8), 1)
    mul = (lane8 < 3).astype(jnp.float32)
    agg8 = w * mul / cnt
    xcv = xc8[...] + agg8
    vz = _silu(jnp.dot(hv, vw1[...], preferred_element_type=jnp.float32) + vb1[...][0:1])
    vw8 = jnp.dot(vz, vw2p[...], preferred_element_type=jnp.float32) + vb2p[...][0:1]
    xcv = xcv + vw8[:, 0:1] * vel8[...]
    xco[...] = xcv + cent8[...]
    magg = outm[...]
    z = _silu(jnp.dot(hv, nw1a[...], preferred_element_type=jnp.float32)
              + jnp.dot(magg, nw1b[...], preferred_element_type=jnp.float32)
              + nb1[...][0:1])
    ho[...] = jnp.dot(z, nw2[...], preferred_element_type=jnp.float32) + nb2[...][0:1]


def _node_call(h, xc8, vel8, outm, outw, cent8,
               vw1, vb1, vw2p, vb2p, nw1a, nw1b, nb1, nw2, nb2):
    nblk = N // NB
    wspec = lambda shp: pl.BlockSpec(shp, lambda i: (0, 0))
    return pl.pallas_call(
        _node_body,
        grid=(nblk,),
        in_specs=[
            pl.BlockSpec((NB, HID), lambda i: (i, 0)),
            pl.BlockSpec((NB, 8), lambda i: (i, 0)),
            pl.BlockSpec((NB, 8), lambda i: (i, 0)),
            pl.BlockSpec((NB, HID), lambda i: (i, 0)),
            pl.BlockSpec((NB, 8), lambda i: (i, 0)),
            pl.BlockSpec((NB, 8), lambda i: (i, 0)),
            wspec((HID, HID)), wspec((8, HID)), wspec((HID, 8)),
            wspec((8, 8)), wspec((HID, HID)), wspec((HID, HID)),
            wspec((8, HID)), wspec((HID, HID)), wspec((8, HID)),
        ],
        out_specs=[
            pl.BlockSpec((NB, HID), lambda i: (i, 0)),
            pl.BlockSpec((NB, 8), lambda i: (i, 0)),
        ],
        out_shape=[jax.ShapeDtypeStruct((N, HID), jnp.float32),
                   jax.ShapeDtypeStruct((N, 8), jnp.float32)],
    )(h, xc8, vel8, outm, outw, cent8,
      vw1, vb1, vw2p, vb2p, nw1a, nw1b, nb1, nw2, nb2)


def _pro_body(h8, x16, embw, embb, ic, cm, h0_o, xcc_o, cent_o):
    h0_o[...] = jnp.dot(h8[...], embw[...], preferred_element_type=jnp.float32) + embb[...][0:1]
    xv = x16[...]
    xcc_o[...] = jnp.dot(xv, ic[...], preferred_element_type=jnp.float32)
    cent_o[...] = jnp.dot(xv, cm[...], preferred_element_type=jnp.float32)


def _pro_call(h8, x16, embw, embb, ic, cm):
    nblk = 100
    wspec = lambda shp: pl.BlockSpec(shp, lambda i: (0, 0))
    return pl.pallas_call(
        _pro_body,
        grid=(nblk,),
        in_specs=[
            pl.BlockSpec((N // nblk, 8), lambda i: (i, 0)),
            pl.BlockSpec((N // G // nblk, 16), lambda i: (i, 0)),
            wspec((8, HID)), wspec((8, HID)), wspec((16, 16)), wspec((16, 16)),
        ],
        out_specs=[
            pl.BlockSpec((N // nblk, HID), lambda i: (i, 0)),
            pl.BlockSpec((N // G // nblk, 16), lambda i: (i, 0)),
            pl.BlockSpec((N // G // nblk, 16), lambda i: (i, 0)),
        ],
        out_shape=[jax.ShapeDtypeStruct((N, HID), jnp.float32),
                   jax.ShapeDtypeStruct((N // G, 16), jnp.float32),
                   jax.ShapeDtypeStruct((N // G, 16), jnp.float32)],
    )(h8, x16, embw, embb, ic, cm)


def _row8(b):
    out = jnp.zeros((8, b.shape[-1]), jnp.float32)
    return out.at[0].set(b)


def kernel(h, x, edges, vel, edge_attr, params):
    row = edges[0]
    col = edges[1]
    row2d = row

    h8 = jnp.pad(h, ((0, 0), (0, 8 - h.shape[1])))
    x16 = jnp.pad(x.reshape(N // G, 3 * G), ((0, 0), (0, 1)))
    vel8 = jnp.pad(vel, ((0, 0), (0, 5)))

    embw = jnp.pad(params['emb_W'], ((0, 8 - params['emb_W'].shape[0]), (0, 0)))
    embb = _row8(params['emb_b'])
    cmat = np.zeros((16, 16), np.float32)
    for i in range(3 * G):
        for j in range(3 * G):
            if i % 3 == j % 3:
                cmat[i, j] = 1.0 / G
    icmat = np.eye(16, dtype=np.float32)
    icmat[15, 15] = 0.0
    icmat = icmat - cmat
    cmat = jnp.asarray(cmat)
    icmat = jnp.asarray(icmat)

    H, xcc16, cent16 = _pro_call(h8, x16, embw, embb, icmat, cmat)
    xc8 = jnp.pad(xcc16[:, :3 * G].reshape(N, 3), ((0, 0), (0, 5)))
    cent8 = jnp.pad(cent16[:, :3 * G].reshape(N, 3), ((0, 0), (0, 5)))
    zcent8 = jnp.zeros_like(cent8)

    zm = jnp.zeros((ZCH, HID), jnp.float32)
    zw = jnp.zeros((ZCH, 8), jnp.float32)

    i4 = jnp.eye(4, dtype=jnp.float32)
    m3c32 = jnp.asarray((np.arange(32) < 3).astype(np.float32).reshape(32, 1))
    dmnp = np.zeros((128, 128), np.float32)
    for g in range(4):
        for k in range(3):
            dmnp[32 * g + k, 32 * g + k] = 1.0
            dmnp[32 * g + k + 8, 32 * g + k] = -1.0
    dm = jnp.asarray(dmnp)
    b3 = jnp.zeros((32, 32), jnp.float32).at[0, 0:3].set(1.0)
    sp = jnp.kron(i4, b3)
    zp16 = jnp.zeros((BLK, 16), jnp.float32)
    _r8w = lambda b: jnp.zeros((8, 128), jnp.float32).at[0].set(jnp.tile(b, 4))

    for li, lp in enumerate(params['layers']):
        bdw1a = jnp.kron(i4, lp['edge_W1'][0:HID])
        bdw1b = jnp.kron(i4, lp['edge_W1'][HID:2 * HID])
        wr = jnp.kron(i4, m3c32 @ lp['edge_W1'][2 * HID:2 * HID + 1])
        wea = jnp.kron(i4, jnp.zeros((HID, HID), jnp.float32
                                     ).at[16:20].set(lp['edge_W1'][2 * HID + 1:]))
        bdw2 = jnp.kron(i4, lp['edge_W2'])
        bdcw1 = jnp.kron(i4, lp['coord_W1'])
        bdcw2 = jnp.kron(i4, jnp.pad(lp['coord_W2'], ((0, 0), (0, 31))))
        vw2p = jnp.pad(lp['vel_W2'], ((0, 0), (0, 7)))
        vb2p = jnp.zeros((8, 8), jnp.float32).at[0, 0].set(lp['vel_b2'][0])
        nw1a = lp['node_W1'][0:HID]
        nw1b = lp['node_W1'][HID:]

        hrow, hcol, rec = _gather_call(H, xc8, row, col, edge_attr, zp16)
        mP, wdP = _edge_call(hrow.reshape(E // 4, 128), hcol.reshape(E // 4, 128),
                             rec.reshape(E // 4, 128),
                             dm, bdw1a, bdw1b, wr, wea, _r8w(lp['edge_b1']),
                             bdw2, _r8w(lp['edge_b2']), bdcw1,
                             _r8w(lp['coord_b1']), bdcw2, sp)
        m = mP.reshape(E, HID)
        wd4 = wdP.reshape(E, HID)
        outm, outw = _scatter_call(m, wd4, row2d, zm, zw)
        outm = jnp.concatenate([outm[:HALF], outm[AROWS:AROWS + HALF]], axis=0)
        outw = jnp.concatenate([outw[:HALF], outw[AROWS:AROWS + HALF]], axis=0)
        H, xc8 = _node_call(H, xc8, vel8, outm, outw,
                            cent8 if li == len(params['layers']) - 1 else zcent8,
                            lp['vel_W1'], _row8(lp['vel_b1']), vw2p, vb2p,
                            nw1a, nw1b, _row8(lp['node_b1']),
                            lp['node_W2'], _row8(lp['node_b2']))

    return xc8[:, :3]


# speedup vs baseline: 2.4184x; 1.0323x over previous
import functools

import jax
import jax.numpy as jnp
import numpy as np
from jax import lax
from jax.experimental import pallas as pl
from jax.experimental.pallas import tpu as pltpu
from jax.experimental.pallas import tpu_sc as plsc

N = 100000
E = 1600000
HID = 32
G = 5
HALF = N // 2
AROWS = 50016
NC, NS, LANES = 2, 16, 16
NW = NC * NS

BLK = 400
SUB = 80
NSUB = BLK // SUB
EPT_G = E // NW
NBLK_G = EPT_G // BLK
SUBB = 128
NBATCH = E // SUBB
QMAX = -(-NBATCH // NS)
ZCH = 521
RPT = AROWS // NS
NZ = RPT // ZCH
DCH = 625
DNZ = (HALF // NS) // DCH

BE = 12800
NB = 2000

_SC_PARAMS = pltpu.CompilerParams(use_tc_tiling_on_sc=False)


def _silu(v):
    return v * jax.nn.sigmoid(v)


def _fire_gathers(h_hbm, xc_hbm, idxr, idxc, hr, hc, xr, xcv, s0, s1, s2, s3):
    for t in range(NSUB):
        sl = pl.ds(t * SUB, SUB)
        pltpu.async_copy(h_hbm.at[idxr.at[sl]], hr.at[sl], s0)
        pltpu.async_copy(h_hbm.at[idxc.at[sl]], hc.at[sl], s1)
        pltpu.async_copy(xc_hbm.at[idxr.at[sl]], xr.at[sl], s2)
        pltpu.async_copy(xc_hbm.at[idxc.at[sl]], xcv.at[sl], s3)


def _wait_gathers(h_hbm, xc_hbm, idxr, idxc, hr, hc, xr, xcv, s0, s1, s2, s3):
    for t in range(NSUB):
        sl = pl.ds(t * SUB, SUB)
        pltpu.make_async_copy(h_hbm.at[idxr.at[sl]], hr.at[sl], s0).wait()
        pltpu.make_async_copy(h_hbm.at[idxc.at[sl]], hc.at[sl], s1).wait()
        pltpu.make_async_copy(xc_hbm.at[idxr.at[sl]], xr.at[sl], s2).wait()
        pltpu.make_async_copy(xc_hbm.at[idxc.at[sl]], xcv.at[sl], s3).wait()


def _write_dsts(base, hrow_hbm, hcol_hbm, rec_hbm):
    return [
        hrow_hbm.at[pl.ds(base, BLK)],
        hcol_hbm.at[pl.ds(base, BLK)],
        rec_hbm.at[pl.ds(base, BLK), pl.ds(0, 8)],
        rec_hbm.at[pl.ds(base, BLK), pl.ds(8, 8)],
        rec_hbm.at[pl.ds(base, BLK), pl.ds(16, 16)],
    ]


def _gather_body(h_hbm, xc_hbm, row_hbm, col_hbm, ea_hbm, zp_hbm,
                 hrow_hbm, hcol_hbm, rec_hbm,
                 idxrA, idxcA, hrA, hcA, xrA, xcvA, eazA,
                 idxrB, idxcB, hrB, hcB, xrB, xcvB, eazB,
                 sA0, sA1, sA2, sA3, sB0, sB1, sB2, sB3, wsA, wsB):
    c = lax.axis_index("c")
    s = lax.axis_index("s")
    wid = s * NC + c
    pltpu.sync_copy(zp_hbm, eazA)
    pltpu.sync_copy(zp_hbm, eazB)
    bufsA = (idxrA, idxcA, hrA, hcA, xrA, xcvA, eazA)
    bufsB = (idxrB, idxcB, hrB, hcB, xrB, xcvB, eazB)
    semsA = (sA0, sA1, sA2, sA3)
    semsB = (sB0, sB1, sB2, sB3)

    def load_and_fire(base, bufs, sems):
        idxr, idxc, hr, hc, xr, xcv, eaz = bufs
        pltpu.sync_copy(row_hbm.at[pl.ds(base, BLK)], idxr)
        pltpu.sync_copy(col_hbm.at[pl.ds(base, BLK)], idxc)
        pltpu.sync_copy(ea_hbm.at[pl.ds(base, BLK)],
                        eaz.at[pl.ds(0, BLK), pl.ds(0, 4)])
        _fire_gathers(h_hbm, xc_hbm, idxr, idxc, hr, hc, xr, xcv, *sems)

    def finish(base, bufs, sems, wsem):
        idxr, idxc, hr, hc, xr, xcv, eaz = bufs
        _wait_gathers(h_hbm, xc_hbm, idxr, idxc, hr, hc, xr, xcv, *sems)
        srcs = [hr, hc, xr, xcv, eaz]
        for sref, dref in zip(srcs, _write_dsts(base, hrow_hbm, hcol_hbm, rec_hbm)):
            pltpu.async_copy(sref, dref, wsem)

    def wait_writes(base, bufs, wsem):
        idxr, idxc, hr, hc, xr, xcv, eaz = bufs
        srcs = [hr, hc, xr, xcv, eaz]
        for sref, dref in zip(srcs, _write_dsts(base, hrow_hbm, hcol_hbm, rec_hbm)):
            pltpu.make_async_copy(sref, dref, wsem).wait()

    npair = NBLK_G // 2

    def pair(k, carry):
        baseA = wid * EPT_G + (2 * k) * BLK
        baseB = wid * EPT_G + (2 * k + 1) * BLK

        @pl.when(k > 0)
        def _():
            wait_writes(baseA, bufsA, wsA)

        load_and_fire(baseA, bufsA, semsA)

        @pl.when(k > 0)
        def _():
            wait_writes(baseB, bufsB, wsB)

        load_and_fire(baseB, bufsB, semsB)
        finish(baseA, bufsA, semsA, wsA)
        finish(baseB, bufsB, semsB, wsB)
        return carry

    lax.fori_loop(0, npair, pair, 0)
    lastA = wid * EPT_G + (2 * npair - 2) * BLK
    lastB = wid * EPT_G + (2 * npair - 1) * BLK
    wait_writes(lastA, bufsA, wsA)
    wait_writes(lastB, bufsB, wsB)
    baseT = wid * EPT_G + (2 * npair) * BLK
    load_and_fire(baseT, bufsA, semsA)
    finish(baseT, bufsA, semsA, wsA)
    wait_writes(baseT, bufsA, wsA)


def _gather_call(h, xc8, row1d, col1d, ea, zp16):
    mesh = plsc.VectorSubcoreMesh(core_axis_name="c", subcore_axis_name="s",
                                  num_cores=NC, num_subcores=NS)
    dbl = lambda: [
        pltpu.VMEM((BLK,), jnp.int32),
        pltpu.VMEM((BLK,), jnp.int32),
        pltpu.VMEM((BLK, HID), jnp.float32),
        pltpu.VMEM((BLK, HID), jnp.float32),
        pltpu.VMEM((BLK, 8), jnp.float32),
        pltpu.VMEM((BLK, 8), jnp.float32),
        pltpu.VMEM((BLK, 16), jnp.float32),
    ]
    f = pl.kernel(
        _gather_body,
        out_type=[jax.ShapeDtypeStruct((E, HID), jnp.float32),
                  jax.ShapeDtypeStruct((E, HID), jnp.float32),
                  jax.ShapeDtypeStruct((E, HID), jnp.float32)],
        mesh=mesh,
        scratch_types=dbl() + dbl() + [pltpu.SemaphoreType.DMA] * 10,
        compiler_params=_SC_PARAMS,
    )
    return f(h, xc8, row1d, col1d, ea, zp16)


def _scatter_body(m_hbm, wd_hbm, row2d_hbm, zm_hbm, zw_hbm,
                  outm_hbm, outw_hbm,
                  accm, accw, idx, mv, wv):
    c = lax.axis_index("c")
    s = lax.axis_index("s")

    for q in range(NZ):
        r = s * RPT + q * ZCH
        pltpu.sync_copy(zm_hbm, accm.at[pl.ds(r, ZCH)])
        pltpu.sync_copy(zw_hbm, accw.at[pl.ds(r, ZCH)])
    plsc.subcore_barrier()

    def blk(q, carry):
        b = q * NS + s

        @pl.when(b < NBATCH)
        def _():
            base = b * SUBB
            pltpu.sync_copy(row2d_hbm.at[pl.ds(base, SUBB)], idx)
            for g in range(SUBB // LANES):
                sl = pl.ds(g * LANES, LANES)
                v = idx[sl]
                lv = v - c * HALF
                ok = (lv >= 0) & (lv < HALF)
                idx[sl] = jnp.where(ok, lv, HALF)
            pltpu.sync_copy(m_hbm.at[pl.ds(base, SUBB)], mv)
            pltpu.sync_copy(wd_hbm.at[pl.ds(base, SUBB), pl.ds(0, 8)], wv)
            pltpu.sync_copy(mv, accm.at[idx], add=True)
            pltpu.sync_copy(wv, accw.at[idx], add=True)

        return carry

    lax.fori_loop(0, QMAX, blk, 0)
    plsc.subcore_barrier()

    for q in range(DNZ):
        r = s * (HALF // NS) + q * DCH
        pltpu.sync_copy(accm.at[pl.ds(r, DCH)], outm_hbm.at[pl.ds(c * HALF + r, DCH)])
        pltpu.sync_copy(accw.at[pl.ds(r, DCH)], outw_hbm.at[pl.ds(c * HALF + r, DCH)])


def _scatter_call(m, wd4, row2d, zm, zw):
    mesh = plsc.VectorSubcoreMesh(core_axis_name="c", subcore_axis_name="s",
                                  num_cores=NC, num_subcores=NS)
    f = pl.kernel(
        _scatter_body,
        out_type=[jax.ShapeDtypeStruct((N, HID), jnp.float32),
                  jax.ShapeDtypeStruct((N, 8), jnp.float32)],
        mesh=mesh,
        scratch_types=[
            pltpu.VMEM_SHARED((AROWS, HID), jnp.float32),
            pltpu.VMEM_SHARED((AROWS, 8), jnp.float32),
            pltpu.VMEM((SUBB,), jnp.int32),
            pltpu.VMEM((SUBB, HID), jnp.float32),
            pltpu.VMEM((SUBB, 8), jnp.float32),
        ],
        compiler_params=_SC_PARAMS,
    )
    return f(m, wd4, row2d, zm, zw)


def _edge_body(hrowP, hcolP, recP,
               dm, bdw1a, bdw1b, wr, wea, b1t, bdw2, b2t, bdcw1, cb1t,
               bdcw2, sp, m_o, wd_o):
    f32 = jnp.float32
    dot = lambda a, b: jnp.dot(a, b, preferred_element_type=f32)
    rec = recP[...]
    d = dot(rec, dm[...])
    dsq = d * d
    z1 = (dot(hrowP[...], bdw1a[...]) + dot(hcolP[...], bdw1b[...])
          + dot(dsq, wr[...]) + dot(rec, wea[...]) + b1t[...][0:1])
    a = _silu(z1)
    m = _silu(dot(a, bdw2[...]) + b2t[...][0:1])
    p = _silu(dot(m, bdcw1[...]) + cb1t[...][0:1])
    cw4 = dot(p, bdcw2[...])
    m_o[...] = m
    cwsp = dot(cw4, sp[...])
    l128 = lax.broadcasted_iota(jnp.int32, (1, 128), 1)
    wd_o[...] = d * cwsp + ((l128 % 32) == 3).astype(f32)


def _edge_call(hrowP, hcolP, recP,
               dm, bdw1a, bdw1b, wr, wea, b1t, bdw2, b2t, bdcw1, cb1t,
               bdcw2, sp):
    nblk = E // BE
    wspec = lambda shp: pl.BlockSpec(shp, lambda i: (0, 0))
    bigspec = pl.BlockSpec((BE // 4, 128), lambda i: (i, 0))
    return pl.pallas_call(
        _edge_body,
        grid=(nblk,),
        in_specs=[
            bigspec, bigspec, bigspec,
            wspec((128, 128)), wspec((128, 128)), wspec((128, 128)),
            wspec((128, 128)), wspec((128, 128)), wspec((8, 128)),
            wspec((128, 128)), wspec((8, 128)), wspec((128, 128)),
            wspec((8, 128)), wspec((128, 128)), wspec((128, 128)),
        ],
        out_specs=[bigspec, bigspec],
        out_shape=[jax.ShapeDtypeStruct((E // 4, 128), jnp.float32),
                   jax.ShapeDtypeStruct((E // 4, 128), jnp.float32)],
    )(hrowP, hcolP, recP,
      dm, bdw1a, bdw1b, wr, wea, b1t, bdw2, b2t, bdcw1, cb1t, bdcw2, sp)


def _node_body(h, xc8, vel8, outm, outw, cent8,
               vw1, vb1, vw2p, vb2p, nw1a, nw1b, nb1, nw2, nb2,
               ho, xco):
    hv = h[...]
    w = outw[...]
    cnt = jnp.maximum(w[:, 3:4], 1.0)
    lane8 = lax.broadcasted_iota(jnp.int32, (1, 8), 1)
    mul = (lane8 < 3).astype(jnp.float32)
    agg8 = w * mul / cnt
    xcv = xc8[...] + agg8
    vz = _silu(jnp.dot(hv, vw1[...], preferred_element_type=jnp.float32) + vb1[...][0:1])
    vw8 = jnp.dot(vz, vw2p[...], preferred_element_type=jnp.float32) + vb2p[...][0:1]
    xcv = xcv + vw8[:, 0:1] * vel8[...]
    xco[...] = xcv + cent8[...]
    magg = outm[...]
    z = _silu(jnp.dot(hv, nw1a[...], preferred_element_type=jnp.float32)
              + jnp.dot(magg, nw1b[...], preferred_element_type=jnp.float32)
              + nb1[...][0:1])
    ho[...] = jnp.dot(z, nw2[...], preferred_element_type=jnp.float32) + nb2[...][0:1]


def _node_call(h, xc8, vel8, outm, outw, cent8,
               vw1, vb1, vw2p, vb2p, nw1a, nw1b, nb1, nw2, nb2):
    nblk = N // NB
    wspec = lambda shp: pl.BlockSpec(shp, lambda i: (0, 0))
    return pl.pallas_call(
        _node_body,
        grid=(nblk,),
        in_specs=[
            pl.BlockSpec((NB, HID), lambda i: (i, 0)),
            pl.BlockSpec((NB, 8), lambda i: (i, 0)),
            pl.BlockSpec((NB, 8), lambda i: (i, 0)),
            pl.BlockSpec((NB, HID), lambda i: (i, 0)),
            pl.BlockSpec((NB, 8), lambda i: (i, 0)),
            pl.BlockSpec((NB, 8), lambda i: (i, 0)),
            wspec((HID, HID)), wspec((8, HID)), wspec((HID, 8)),
            wspec((8, 8)), wspec((HID, HID)), wspec((HID, HID)),
            wspec((8, HID)), wspec((HID, HID)), wspec((8, HID)),
        ],
        out_specs=[
            pl.BlockSpec((NB, HID), lambda i: (i, 0)),
            pl.BlockSpec((NB, 8), lambda i: (i, 0)),
        ],
        out_shape=[jax.ShapeDtypeStruct((N, HID), jnp.float32),
                   jax.ShapeDtypeStruct((N, 8), jnp.float32)],
    )(h, xc8, vel8, outm, outw, cent8,
      vw1, vb1, vw2p, vb2p, nw1a, nw1b, nb1, nw2, nb2)


def _pro_body(h8, x16, embw, embb, ic, cm, h0_o, xcc_o, cent_o):
    h0_o[...] = jnp.dot(h8[...], embw[...], preferred_element_type=jnp.float32) + embb[...][0:1]
    xv = x16[...]
    xcc_o[...] = jnp.dot(xv, ic[...], preferred_element_type=jnp.float32)
    cent_o[...] = jnp.dot(xv, cm[...], preferred_element_type=jnp.float32)


def _pro_call(h8, x16, embw, embb, ic, cm):
    nblk = 100
    wspec = lambda shp: pl.BlockSpec(shp, lambda i: (0, 0))
    return pl.pallas_call(
        _pro_body,
        grid=(nblk,),
        in_specs=[
            pl.BlockSpec((N // nblk, 8), lambda i: (i, 0)),
            pl.BlockSpec((N // G // nblk, 16), lambda i: (i, 0)),
            wspec((8, HID)), wspec((8, HID)), wspec((16, 16)), wspec((16, 16)),
        ],
        out_specs=[
            pl.BlockSpec((N // nblk, HID), lambda i: (i, 0)),
            pl.BlockSpec((N // G // nblk, 16), lambda i: (i, 0)),
            pl.BlockSpec((N // G // nblk, 16), lambda i: (i, 0)),
        ],
        out_shape=[jax.ShapeDtypeStruct((N, HID), jnp.float32),
                   jax.ShapeDtypeStruct((N // G, 16), jnp.float32),
                   jax.ShapeDtypeStruct((N // G, 16), jnp.float32)],
    )(h8, x16, embw, embb, ic, cm)


def _row8(b):
    out = jnp.zeros((8, b.shape[-1]), jnp.float32)
    return out.at[0].set(b)


def kernel(h, x, edges, vel, edge_attr, params):
    row = edges[0]
    col = edges[1]
    row2d = row

    h8 = jnp.pad(h, ((0, 0), (0, 8 - h.shape[1])))
    x16 = jnp.pad(x.reshape(N // G, 3 * G), ((0, 0), (0, 1)))
    vel8 = jnp.pad(vel, ((0, 0), (0, 5)))

    embw = jnp.pad(params['emb_W'], ((0, 8 - params['emb_W'].shape[0]), (0, 0)))
    embb = _row8(params['emb_b'])
    cmat = np.zeros((16, 16), np.float32)
    for i in range(3 * G):
        for j in range(3 * G):
            if i % 3 == j % 3:
                cmat[i, j] = 1.0 / G
    icmat = np.eye(16, dtype=np.float32)
    icmat[15, 15] = 0.0
    icmat = icmat - cmat
    cmat = jnp.asarray(cmat)
    icmat = jnp.asarray(icmat)

    H, xcc16, cent16 = _pro_call(h8, x16, embw, embb, icmat, cmat)
    xc8 = jnp.pad(xcc16[:, :3 * G].reshape(N, 3), ((0, 0), (0, 5)))
    cent8 = jnp.pad(cent16[:, :3 * G].reshape(N, 3), ((0, 0), (0, 5)))
    zcent8 = jnp.zeros_like(cent8)

    zm = jnp.zeros((ZCH, HID), jnp.float32)
    zw = jnp.zeros((ZCH, 8), jnp.float32)

    i4 = jnp.eye(4, dtype=jnp.float32)
    m3c32 = jnp.asarray((np.arange(32) < 3).astype(np.float32).reshape(32, 1))
    dmnp = np.zeros((128, 128), np.float32)
    for g in range(4):
        for k in range(3):
            dmnp[32 * g + k, 32 * g + k] = 1.0
            dmnp[32 * g + k + 8, 32 * g + k] = -1.0
    dm = jnp.asarray(dmnp)
    b3 = jnp.zeros((32, 32), jnp.float32).at[0, 0:3].set(1.0)
    sp = jnp.kron(i4, b3)
    zp16 = jnp.zeros((BLK, 16), jnp.float32)
    _r8w = lambda b: jnp.zeros((8, 128), jnp.float32).at[0].set(jnp.tile(b, 4))

    for li, lp in enumerate(params['layers']):
        bdw1a = jnp.kron(i4, lp['edge_W1'][0:HID])
        bdw1b = jnp.kron(i4, lp['edge_W1'][HID:2 * HID])
        wr = jnp.kron(i4, m3c32 @ lp['edge_W1'][2 * HID:2 * HID + 1])
        wea = jnp.kron(i4, jnp.zeros((HID, HID), jnp.float32
                                     ).at[16:20].set(lp['edge_W1'][2 * HID + 1:]))
        bdw2 = jnp.kron(i4, lp['edge_W2'])
        bdcw1 = jnp.kron(i4, lp['coord_W1'])
        bdcw2 = jnp.kron(i4, jnp.pad(lp['coord_W2'], ((0, 0), (0, 31))))
        vw2p = jnp.pad(lp['vel_W2'], ((0, 0), (0, 7)))
        vb2p = jnp.zeros((8, 8), jnp.float32).at[0, 0].set(lp['vel_b2'][0])
        nw1a = lp['node_W1'][0:HID]
        nw1b = lp['node_W1'][HID:]

        hrow, hcol, rec = _gather_call(H, xc8, row, col, edge_attr, zp16)
        mP, wdP = _edge_call(hrow.reshape(E // 4, 128), hcol.reshape(E // 4, 128),
                             rec.reshape(E // 4, 128),
                             dm, bdw1a, bdw1b, wr, wea, _r8w(lp['edge_b1']),
                             bdw2, _r8w(lp['edge_b2']), bdcw1,
                             _r8w(lp['coord_b1']), bdcw2, sp)
        m = mP.reshape(E, HID)
        wd4 = wdP.reshape(E, HID)
        outm, outw = _scatter_call(m, wd4, row2d, zm, zw)
        H, xc8 = _node_call(H, xc8, vel8, outm, outw,
                            cent8 if li == len(params['layers']) - 1 else zcent8,
                            lp['vel_W1'], _row8(lp['vel_b1']), vw2p, vb2p,
                            nw1a, nw1b, _row8(lp['node_b1']),
                            lp['node_W2'], _row8(lp['node_b2']))

    return xc8[:, :3]
